# Initial kernel scaffold; baseline (speedup 1.0000x reference)
import jax
import jax.numpy as jnp
from jax.experimental import pallas as pl


def kernel(x, edge_index, batch, params):
    raise NotImplementedError("write your pallas kernel here")


# speedup vs baseline: 11.1718x; 11.1718x over previous
import functools

import jax
import jax.numpy as jnp
from jax import lax
from jax.experimental import pallas as pl
from jax.experimental.pallas import tpu as pltpu
from jax.experimental.pallas import tpu_sc as plsc

N = 50000
E = 800000
IN = 128
D = 64
G = 64
L = 3

NP = 51200
EP = 819200
BR = 1024
GRID = NP // BR

F32 = jnp.float32


def _in_proj_body(x_ref, wt_ref, b_ref, o_ref):
    o_ref[...] = jnp.dot(x_ref[...], wt_ref[...],
                         preferred_element_type=F32) + b_ref[...]


def _in_proj(x, wt, b):
    return pl.pallas_call(
        _in_proj_body,
        grid=(GRID,),
        in_specs=[
            pl.BlockSpec((BR, IN), lambda i: (i, 0)),
            pl.BlockSpec((IN, D), lambda i: (0, 0)),
            pl.BlockSpec((D,), lambda i: (0,)),
        ],
        out_specs=pl.BlockSpec((BR, D), lambda i: (i, 0)),
        out_shape=jax.ShapeDtypeStruct((NP, D), F32),
    )(x, wt, b)


def _pre_body(xc_ref, gwt_ref, asrc_ref, adst_ref, hwt_ref, hb_ref,
              xl_ref, s_ref, d_ref, hh_ref):
    xc = xc_ref[...]
    xl = jnp.dot(xc, gwt_ref[...], preferred_element_type=F32)
    xl_ref[0] = xl[:, :32]
    xl_ref[1] = xl[:, 32:]
    s_ref[...] = jnp.sum(xl * asrc_ref[...][None, :], axis=1)
    d_ref[...] = jnp.sum(xl * adst_ref[...][None, :], axis=1)
    hh_ref[...] = jax.nn.relu(
        jnp.dot(xc, hwt_ref[...], preferred_element_type=F32) + hb_ref[...])


def _pre(xc, gwt, asrc, adst, hwt, hb):
    return pl.pallas_call(
        _pre_body,
        grid=(GRID,),
        in_specs=[
            pl.BlockSpec((BR, D), lambda i: (i, 0)),
            pl.BlockSpec((D, D), lambda i: (0, 0)),
            pl.BlockSpec((D,), lambda i: (0,)),
            pl.BlockSpec((D,), lambda i: (0,)),
            pl.BlockSpec((D, D), lambda i: (0, 0)),
            pl.BlockSpec((D,), lambda i: (0,)),
        ],
        out_specs=[
            pl.BlockSpec((2, BR, 32), lambda i: (0, i, 0)),
            pl.BlockSpec((BR,), lambda i: (i,)),
            pl.BlockSpec((BR,), lambda i: (i,)),
            pl.BlockSpec((BR, D), lambda i: (i, 0)),
        ],
        out_shape=[
            jax.ShapeDtypeStruct((2, NP, 32), F32),
            jax.ShapeDtypeStruct((NP,), F32),
            jax.ShapeDtypeStruct((NP,), F32),
            jax.ShapeDtypeStruct((NP, D), F32),
        ],
    )(xc, gwt, asrc, adst, hwt, hb)


def _post_body(conv_ref, xc_ref, hh_ref, xloc_ref, lwt_ref, lb_ref, gb_ref,
               lnw_ref, lnb_ref, beta_ref, xcn_ref, xlocn_ref):
    conv = jnp.concatenate([conv_ref[0], conv_ref[1]], axis=1) + gb_ref[...]
    t = jnp.dot(xc_ref[...], lwt_ref[...],
                preferred_element_type=F32) + lb_ref[...] + conv
    xc2 = jax.nn.relu(t)
    z = hh_ref[...] * xc2
    mu = jnp.mean(z, axis=-1, keepdims=True)
    var = jnp.mean((z - mu) ** 2, axis=-1, keepdims=True)
    ln = (z - mu) / jnp.sqrt(var + 1e-5) * lnw_ref[...] + lnb_ref[...]
    beta = jax.nn.sigmoid(beta_ref[...])[None, :]
    xcn = (1.0 - beta) * ln + beta * xc2
    xcn_ref[...] = xcn
    xlocn_ref[...] = xloc_ref[...] + xcn


def _post(conv, xc, hh, xloc, lwt, lb, gb, lnw, lnb, beta):
    return pl.pallas_call(
        _post_body,
        grid=(GRID,),
        in_specs=[
            pl.BlockSpec((2, BR, 32), lambda i: (0, i, 0)),
            pl.BlockSpec((BR, D), lambda i: (i, 0)),
            pl.BlockSpec((BR, D), lambda i: (i, 0)),
            pl.BlockSpec((BR, D), lambda i: (i, 0)),
            pl.BlockSpec((D, D), lambda i: (0, 0)),
            pl.BlockSpec((D,), lambda i: (0,)),
            pl.BlockSpec((D,), lambda i: (0,)),
            pl.BlockSpec((D,), lambda i: (0,)),
            pl.BlockSpec((D,), lambda i: (0,)),
            pl.BlockSpec((D,), lambda i: (0,)),
        ],
        out_specs=[
            pl.BlockSpec((BR, D), lambda i: (i, 0)),
            pl.BlockSpec((BR, D), lambda i: (i, 0)),
        ],
        out_shape=[
            jax.ShapeDtypeStruct((NP, D), F32),
            jax.ShapeDtypeStruct((NP, D), F32),
        ],
    )(conv, xc, hh, xloc, lwt, lb, gb, lnw, lnb, beta)


def _pool_body(batch_ref, xl_ref, pwt_ref, pb_ref, o_ref, acc_s, acc_c):
    pid = pl.program_id(0)

    @pl.when(pid == 0)
    def _():
        acc_s[...] = jnp.zeros((G, D), F32)
        acc_c[...] = jnp.zeros((G, D), F32)

    b = batch_ref[...]
    gids = lax.broadcasted_iota(jnp.int32, (G, BR), 0)
    mask = (b[None, :] == gids).astype(F32)
    acc_s[...] += jnp.dot(mask, xl_ref[...], preferred_element_type=F32)
    cnt = jnp.sum(mask, axis=1)
    acc_c[...] += jnp.broadcast_to(cnt[:, None], (G, D))

    @pl.when(pid == GRID - 1)
    def _():
        pooled = acc_s[...] / jnp.maximum(acc_c[...], 1.0)
        o_ref[...] = jnp.dot(pooled, pwt_ref[...],
                             preferred_element_type=F32) + pb_ref[...]


def _pool(batch_pad, xloc, pwt, pb):
    return pl.pallas_call(
        _pool_body,
        grid=(GRID,),
        in_specs=[
            pl.BlockSpec((BR,), lambda i: (i,)),
            pl.BlockSpec((BR, D), lambda i: (i, 0)),
            pl.BlockSpec((D, G), lambda i: (0, 0)),
            pl.BlockSpec((G,), lambda i: (0,)),
        ],
        out_specs=pl.BlockSpec((G, G), lambda i: (0, 0)),
        out_shape=jax.ShapeDtypeStruct((G, G), F32),
        scratch_shapes=[pltpu.VMEM((G, D), F32), pltpu.VMEM((G, D), F32)],
    )(batch_pad, xloc, pwt, pb)


_NC = 2
_NS = 16
_CH = 128
_NPT = NP // _NS
_P1_PER = EP // (_NC * _NS)
_P1_CHUNKS = _P1_PER // _CH
_P2_PER = EP // _NS
_P2_CHUNKS = _P2_PER // _CH

_SCMESH = plsc.VectorSubcoreMesh(core_axis_name="c", subcore_axis_name="s",
                                 num_cores=_NC, num_subcores=_NS)


@functools.partial(
    pl.kernel,
    out_type=[jax.ShapeDtypeStruct((EP,), F32),
              jax.ShapeDtypeStruct((_NC, NP), F32)],
    mesh=_SCMESH,
    compiler_params=pltpu.CompilerParams(use_tc_tiling_on_sc=False),
    scratch_types=[
        pltpu.VMEM((_CH,), jnp.int32),
        pltpu.VMEM((_CH,), jnp.int32),
        pltpu.VMEM((_CH,), F32),
        pltpu.VMEM((_CH,), F32),
        pltpu.VMEM((_CH,), F32),
        pltpu.VMEM((_NPT,), F32),
        pltpu.VMEM_SHARED((NP,), F32),
        pltpu.SemaphoreType.DMA,
        pltpu.SemaphoreType.DMA,
    ])
def _edge_pass1(src_hbm, dst_hbm, s_hbm, d_hbm, w_hbm, dpart_hbm,
                srcb, dstb, sv, dv, wv, bounce, den_sh, sem1, sem2):
    cid = lax.axis_index("c")
    sid = lax.axis_index("s")
    wid = cid * _NS + sid
    nbase = sid * _NPT

    zero = jnp.zeros((16,), F32)

    def zbody(i, _):
        bounce[pl.ds(i * 16, 16)] = zero
        return 0

    lax.fori_loop(0, _NPT // 16, zbody, 0)
    pltpu.sync_copy(bounce, den_sh.at[pl.ds(nbase, _NPT)])
    plsc.subcore_barrier()

    ebase0 = wid * _P1_PER

    def body(i, _):
        eb = ebase0 + i * _CH
        pltpu.sync_copy(src_hbm.at[pl.ds(eb, _CH)], srcb)
        pltpu.sync_copy(dst_hbm.at[pl.ds(eb, _CH)], dstb)
        pltpu.async_copy(s_hbm.at[srcb], sv, sem1).wait()
        pltpu.async_copy(d_hbm.at[dstb], dv, sem2).wait()
        for g in range(_CH // 16):
            a = sv[pl.ds(g * 16, 16)] + dv[pl.ds(g * 16, 16)]
            a = jnp.maximum(a, 0.0) + 0.2 * jnp.minimum(a, 0.0)
            wv[pl.ds(g * 16, 16)] = jnp.exp(a)
        pltpu.sync_copy(wv, w_hbm.at[pl.ds(eb, _CH)])
        pltpu.sync_copy(wv, den_sh.at[dstb], add=True)
        return 0

    lax.fori_loop(0, _P1_CHUNKS, body, 0)
    plsc.subcore_barrier()
    pltpu.sync_copy(den_sh.at[pl.ds(nbase, _NPT)], bounce)
    pltpu.sync_copy(bounce, dpart_hbm.at[cid].at[pl.ds(nbase, _NPT)])


@functools.partial(
    pl.kernel,
    out_type=jax.ShapeDtypeStruct((_NC * NP, 32), F32),
    mesh=_SCMESH,
    compiler_params=pltpu.CompilerParams(use_tc_tiling_on_sc=False),
    scratch_types=[
        pltpu.VMEM((_CH,), jnp.int32),
        pltpu.VMEM((_CH,), jnp.int32),
        pltpu.VMEM((_CH,), jnp.int32),
        pltpu.VMEM((_CH,), F32),
        pltpu.VMEM((_CH,), F32),
        pltpu.VMEM((_CH, 32), F32),
        pltpu.VMEM((_CH, 32), F32),
        pltpu.VMEM_SHARED((NP, 32), F32),
        pltpu.SemaphoreType.DMA,
        pltpu.SemaphoreType.DMA,
    ])
def _edge_pass2(src_hbm, dst_hbm, w_hbm, dinv_hbm, zero_hbm, xl_hbm, out_hbm,
                srcb, dstb, srcb2, wv, dvv, rows, msg, acc_sh, sem1, sem2):
    cid = lax.axis_index("c")
    sid = lax.axis_index("s")
    nbase = sid * _NPT

    pltpu.sync_copy(zero_hbm, acc_sh.at[pl.ds(nbase, _NPT), :])
    plsc.subcore_barrier()

    ebase0 = sid * _P2_PER
    rowoff = cid * NP

    def body(i, _):
        eb = ebase0 + i * _CH
        pltpu.sync_copy(src_hbm.at[pl.ds(eb, _CH)], srcb)
        pltpu.sync_copy(dst_hbm.at[pl.ds(eb, _CH)], dstb)
        pltpu.sync_copy(w_hbm.at[pl.ds(eb, _CH)], wv)
        for g in range(_CH // 16):
            srcb2[pl.ds(g * 16, 16)] = srcb[pl.ds(g * 16, 16)] + rowoff
        pltpu.async_copy(xl_hbm.at[srcb2], rows, sem1).wait()
        pltpu.async_copy(dinv_hbm.at[dstb], dvv, sem2).wait()
        for g in range(_CH // 16):
            attg = wv[pl.ds(g * 16, 16)] * dvv[pl.ds(g * 16, 16)]
            for j in range(16):
                e = g * 16 + j
                att = attg[j]
                msg[e, pl.ds(0, 16)] = rows[e, pl.ds(0, 16)] * att
                msg[e, pl.ds(16, 16)] = rows[e, pl.ds(16, 16)] * att
        pltpu.sync_copy(msg, acc_sh.at[dstb], add=True)
        return 0

    lax.fori_loop(0, _P2_CHUNKS, body, 0)
    plsc.subcore_barrier()
    pltpu.sync_copy(acc_sh.at[pl.ds(nbase, _NPT), :],
                    out_hbm.at[pl.ds(rowoff + nbase, _NPT), :])


_Z32 = None


def _gat_edges(src, dst, s, d, xl2):
    w, dpart = _edge_pass1(src, dst, s, d)
    dinv = 1.0 / (dpart[0] + dpart[1] + 1e-16)
    zero32 = jnp.zeros((_NPT, 32), F32)
    out = _edge_pass2(src, dst, w, dinv, zero32, xl2.reshape(_NC * NP, 32))
    return out.reshape(_NC, NP, 32)


def kernel(x, edge_index, batch, params):
    src = jnp.concatenate([edge_index[0],
                           jnp.zeros((EP - E,), jnp.int32)])
    dst = jnp.concatenate([edge_index[1],
                           jnp.full((EP - E,), N, jnp.int32)])
    batch_pad = jnp.concatenate([batch, jnp.full((NP - N,), G, jnp.int32)])
    xp = jnp.pad(x, ((0, NP - N), (0, 0)))

    xc = _in_proj(xp, params['lin_in_W'].T, params['lin_in_b'])
    xloc = jnp.zeros((NP, D), F32)
    for i in range(L):
        p = params['layers'][i]
        xl2, s, d, hh = _pre(xc, p['gat_W'].T, p['att_src'][0, 0],
                             p['att_dst'][0, 0], p['h_W'].T, p['h_b'])
        conv = _gat_edges(src, dst, s, d, xl2)
        xc, xloc = _post(conv, xc, hh, xloc, p['lin_W'].T, p['lin_b'],
                         p['gat_b'], p['ln_w'], p['ln_b'],
                         params['betas'][i])
    return _pool(batch_pad, xloc, params['pred_W'].T, params['pred_b'])


# speedup vs baseline: 21.7758x; 1.9492x over previous
import functools

import jax
import jax.numpy as jnp
from jax import lax
from jax.experimental import pallas as pl
from jax.experimental.pallas import tpu as pltpu
from jax.experimental.pallas import tpu_sc as plsc

N = 50000
E = 800000
IN = 128
D = 64
G = 64
L = 3

NP = 51200
EP = 819200
BR = 1024
GRID = NP // BR

F32 = jnp.float32


def _in_proj_body(x_ref, wt_ref, b_ref, o_ref):
    o_ref[...] = jnp.dot(x_ref[...], wt_ref[...],
                         preferred_element_type=F32) + b_ref[...]


def _in_proj(x, wt, b):
    return pl.pallas_call(
        _in_proj_body,
        grid=(GRID,),
        in_specs=[
            pl.BlockSpec((BR, IN), lambda i: (i, 0)),
            pl.BlockSpec((IN, D), lambda i: (0, 0)),
            pl.BlockSpec((D,), lambda i: (0,)),
        ],
        out_specs=pl.BlockSpec((BR, D), lambda i: (i, 0)),
        out_shape=jax.ShapeDtypeStruct((NP, D), F32),
    )(x, wt, b)


def _pre_body(xc_ref, gwt_ref, asrc_ref, adst_ref, hwt_ref, hb_ref,
              xl_ref, s_ref, d_ref, hh_ref):
    xc = xc_ref[...]
    xl = jnp.dot(xc, gwt_ref[...], preferred_element_type=F32)
    for q in range(4):
        xl_ref[q] = xl[:, 16 * q:16 * q + 16]
    s_ref[...] = jnp.sum(xl * asrc_ref[...][None, :], axis=1)
    d_ref[...] = jnp.sum(xl * adst_ref[...][None, :], axis=1)
    hh_ref[...] = jax.nn.relu(
        jnp.dot(xc, hwt_ref[...], preferred_element_type=F32) + hb_ref[...])


def _pre(xc, gwt, asrc, adst, hwt, hb):
    return pl.pallas_call(
        _pre_body,
        grid=(GRID,),
        in_specs=[
            pl.BlockSpec((BR, D), lambda i: (i, 0)),
            pl.BlockSpec((D, D), lambda i: (0, 0)),
            pl.BlockSpec((D,), lambda i: (0,)),
            pl.BlockSpec((D,), lambda i: (0,)),
            pl.BlockSpec((D, D), lambda i: (0, 0)),
            pl.BlockSpec((D,), lambda i: (0,)),
        ],
        out_specs=[
            pl.BlockSpec((4, BR, 16), lambda i: (0, i, 0)),
            pl.BlockSpec((BR,), lambda i: (i,)),
            pl.BlockSpec((BR,), lambda i: (i,)),
            pl.BlockSpec((BR, D), lambda i: (i, 0)),
        ],
        out_shape=[
            jax.ShapeDtypeStruct((4, NP, 16), F32),
            jax.ShapeDtypeStruct((NP,), F32),
            jax.ShapeDtypeStruct((NP,), F32),
            jax.ShapeDtypeStruct((NP, D), F32),
        ],
    )(xc, gwt, asrc, adst, hwt, hb)


def _post_body(conv_ref, dinv_ref, xc_ref, hh_ref, xloc_ref, lwt_ref, lb_ref,
               gb_ref, lnw_ref, lnb_ref, beta_ref, xcn_ref, xlocn_ref):
    conv = (jnp.concatenate([conv_ref[q] for q in range(4)], axis=1)
            * dinv_ref[...][:, None] + gb_ref[...])
    t = jnp.dot(xc_ref[...], lwt_ref[...],
                preferred_element_type=F32) + lb_ref[...] + conv
    xc2 = jax.nn.relu(t)
    z = hh_ref[...] * xc2
    mu = jnp.mean(z, axis=-1, keepdims=True)
    var = jnp.mean((z - mu) ** 2, axis=-1, keepdims=True)
    ln = (z - mu) / jnp.sqrt(var + 1e-5) * lnw_ref[...] + lnb_ref[...]
    beta = jax.nn.sigmoid(beta_ref[...])[None, :]
    xcn = (1.0 - beta) * ln + beta * xc2
    xcn_ref[...] = xcn
    xlocn_ref[...] = xloc_ref[...] + xcn


def _post(conv, dinv, xc, hh, xloc, lwt, lb, gb, lnw, lnb, beta):
    return pl.pallas_call(
        _post_body,
        grid=(GRID,),
        in_specs=[
            pl.BlockSpec((4, BR, 16), lambda i: (0, i, 0)),
            pl.BlockSpec((BR,), lambda i: (i,)),
            pl.BlockSpec((BR, D), lambda i: (i, 0)),
            pl.BlockSpec((BR, D), lambda i: (i, 0)),
            pl.BlockSpec((BR, D), lambda i: (i, 0)),
            pl.BlockSpec((D, D), lambda i: (0, 0)),
            pl.BlockSpec((D,), lambda i: (0,)),
            pl.BlockSpec((D,), lambda i: (0,)),
            pl.BlockSpec((D,), lambda i: (0,)),
            pl.BlockSpec((D,), lambda i: (0,)),
            pl.BlockSpec((D,), lambda i: (0,)),
        ],
        out_specs=[
            pl.BlockSpec((BR, D), lambda i: (i, 0)),
            pl.BlockSpec((BR, D), lambda i: (i, 0)),
        ],
        out_shape=[
            jax.ShapeDtypeStruct((NP, D), F32),
            jax.ShapeDtypeStruct((NP, D), F32),
        ],
    )(conv, dinv, xc, hh, xloc, lwt, lb, gb, lnw, lnb, beta)


def _pool_body(batch_ref, xl_ref, pwt_ref, pb_ref, o_ref, acc_s, acc_c):
    pid = pl.program_id(0)

    @pl.when(pid == 0)
    def _():
        acc_s[...] = jnp.zeros((G, D), F32)
        acc_c[...] = jnp.zeros((G, D), F32)

    b = batch_ref[...]
    gids = lax.broadcasted_iota(jnp.int32, (G, BR), 0)
    mask = (b[None, :] == gids).astype(F32)
    acc_s[...] += jnp.dot(mask, xl_ref[...], preferred_element_type=F32)
    cnt = jnp.sum(mask, axis=1)
    acc_c[...] += jnp.broadcast_to(cnt[:, None], (G, D))

    @pl.when(pid == GRID - 1)
    def _():
        pooled = acc_s[...] / jnp.maximum(acc_c[...], 1.0)
        o_ref[...] = jnp.dot(pooled, pwt_ref[...],
                             preferred_element_type=F32) + pb_ref[...]


def _pool(batch_pad, xloc, pwt, pb):
    return pl.pallas_call(
        _pool_body,
        grid=(GRID,),
        in_specs=[
            pl.BlockSpec((BR,), lambda i: (i,)),
            pl.BlockSpec((BR, D), lambda i: (i, 0)),
            pl.BlockSpec((D, G), lambda i: (0, 0)),
            pl.BlockSpec((G,), lambda i: (0,)),
        ],
        out_specs=pl.BlockSpec((G, G), lambda i: (0, 0)),
        out_shape=jax.ShapeDtypeStruct((G, G), F32),
        scratch_shapes=[pltpu.VMEM((G, D), F32), pltpu.VMEM((G, D), F32)],
    )(batch_pad, xloc, pwt, pb)


_NC = 2
_NS = 16
_CH = 128
_NPT = NP // _NS
_P1_PER = EP // (_NC * _NS)
_P1_CHUNKS = _P1_PER // _CH
_P2_PER = EP // _NS
_P2_CHUNKS = _P2_PER // _CH

_SCMESH = plsc.VectorSubcoreMesh(core_axis_name="c", subcore_axis_name="s",
                                 num_cores=_NC, num_subcores=_NS)
_K = 4


@functools.partial(
    pl.kernel,
    out_type=[jax.ShapeDtypeStruct((EP,), F32),
              jax.ShapeDtypeStruct((_NC, NP), F32)],
    mesh=_SCMESH,
    compiler_params=pltpu.CompilerParams(use_tc_tiling_on_sc=False),
    scratch_types=[
        pltpu.VMEM((_K, _CH), jnp.int32),
        pltpu.VMEM((_K, _CH), jnp.int32),
        pltpu.VMEM((_K, _CH), F32),
        pltpu.VMEM((_K, _CH), F32),
        pltpu.VMEM((_K, _CH), F32),
        pltpu.VMEM((_NPT,), F32),
        pltpu.VMEM_SHARED((NP,), F32),
        pltpu.SemaphoreType.DMA,
        pltpu.SemaphoreType.DMA,
        pltpu.SemaphoreType.DMA,
        pltpu.SemaphoreType.DMA,
        pltpu.SemaphoreType.DMA,
    ])
def _edge_pass1(src_hbm, dst_hbm, s_hbm, d_hbm, w_hbm, dpart_hbm,
                srcb, dstb, sv, dv, wv, bounce, den_sh,
                semA, semG0, semG1, semG2, semG3):
    cid = lax.axis_index("c")
    sid = lax.axis_index("s")
    wid = cid * _NS + sid
    nbase = sid * _NPT
    semG = [semG0, semG1, semG2, semG3]

    zero = jnp.zeros((16,), F32)

    def zbody(i, _):
        bounce[pl.ds(i * 16, 16)] = zero
        return 0

    lax.fori_loop(0, _NPT // 16, zbody, 0)
    pltpu.sync_copy(bounce, den_sh.at[pl.ds(nbase, _NPT)])
    plsc.subcore_barrier()

    ebase0 = wid * _P1_PER

    def body(i, _):
        gb = ebase0 + i * (_K * _CH)
        loads = []
        for j in range(_K):
            eb = gb + j * _CH
            loads.append(pltpu.async_copy(
                src_hbm.at[pl.ds(eb, _CH)], srcb.at[j], semA))
            loads.append(pltpu.async_copy(
                dst_hbm.at[pl.ds(eb, _CH)], dstb.at[j], semA))
        for cp in loads:
            cp.wait()
        gcps = []
        for j in range(_K):
            c1 = pltpu.async_copy(s_hbm.at[srcb.at[j]], sv.at[j], semG[j])
            c2 = pltpu.async_copy(d_hbm.at[dstb.at[j]], dv.at[j], semG[j])
            gcps.append((c1, c2))
        for j in range(_K):
            gcps[j][0].wait()
            gcps[j][1].wait()
            for g in range(_CH // 16):
                a = sv[j, pl.ds(g * 16, 16)] + dv[j, pl.ds(g * 16, 16)]
                a = jnp.maximum(a, 0.0) + 0.2 * jnp.minimum(a, 0.0)
                wv[j, pl.ds(g * 16, 16)] = jnp.exp(a)
            pltpu.sync_copy(wv.at[j], w_hbm.at[pl.ds(gb + j * _CH, _CH)])
            pltpu.sync_copy(wv.at[j], den_sh.at[dstb.at[j]], add=True)
        return 0

    lax.fori_loop(0, _P1_CHUNKS // _K, body, 0)
    plsc.subcore_barrier()
    pltpu.sync_copy(den_sh.at[pl.ds(nbase, _NPT)], bounce)
    pltpu.sync_copy(bounce, dpart_hbm.at[cid].at[pl.ds(nbase, _NPT)])


@functools.partial(
    pl.kernel,
    out_type=jax.ShapeDtypeStruct((4 * NP, 16), F32),
    mesh=_SCMESH,
    compiler_params=pltpu.CompilerParams(use_tc_tiling_on_sc=False),
    scratch_types=[
        pltpu.VMEM((_K, _CH), jnp.int32),
        pltpu.VMEM((_K, _CH), jnp.int32),
        pltpu.VMEM((_K, _CH), jnp.int32),
        pltpu.VMEM((_K, _CH), F32),
        pltpu.VMEM((_K, _CH, 16), F32),
        pltpu.VMEM((_K, _CH, 16), F32),
        pltpu.VMEM_SHARED((NP, 16), F32),
        pltpu.SemaphoreType.DMA,
        pltpu.SemaphoreType.DMA,
        pltpu.SemaphoreType.DMA,
        pltpu.SemaphoreType.DMA,
        pltpu.SemaphoreType.DMA,
    ])
def _edge_pass2(src_hbm, dst_hbm, w_hbm, zero_hbm, xl_hbm, out_hbm,
                srcb, dstb, srcb2, wv, rows, msg, acc_sh,
                semA, semG0, semG1, semG2, semG3):
    cid = lax.axis_index("c")
    sid = lax.axis_index("s")
    nbase = sid * _NPT
    semG = [semG0, semG1, semG2, semG3]
    ebase0 = sid * _P2_PER

    for sweep in range(2):
        rowoff = (2 * sweep + cid) * NP

        pltpu.sync_copy(zero_hbm, acc_sh.at[pl.ds(nbase, _NPT), :])
        plsc.subcore_barrier()

        def body(i, _):
            gb = ebase0 + i * (_K * _CH)
            loads = []
            for j in range(_K):
                eb = gb + j * _CH
                loads.append(pltpu.async_copy(
                    src_hbm.at[pl.ds(eb, _CH)], srcb.at[j], semA))
                loads.append(pltpu.async_copy(
                    dst_hbm.at[pl.ds(eb, _CH)], dstb.at[j], semA))
                loads.append(pltpu.async_copy(
                    w_hbm.at[pl.ds(eb, _CH)], wv.at[j], semA))
            for cp in loads:
                cp.wait()
            gcps = []
            for j in range(_K):
                for g in range(_CH // 16):
                    srcb2[j, pl.ds(g * 16, 16)] = (
                        srcb[j, pl.ds(g * 16, 16)] + rowoff)
                gcps.append(pltpu.async_copy(
                    xl_hbm.at[srcb2.at[j]], rows.at[j], semG[j]))
            for j in range(_K):
                gcps[j].wait()
                for g in range(_CH // 16):
                    attg = wv[j, pl.ds(g * 16, 16)]
                    for t in range(16):
                        e = g * 16 + t
                        msg[j, e, pl.ds(0, 16)] = (
                            rows[j, e, pl.ds(0, 16)] * attg[t])
                pltpu.sync_copy(msg.at[j], acc_sh.at[dstb.at[j]], add=True)
            return 0

        lax.fori_loop(0, _P2_CHUNKS // _K, body, 0)
        plsc.subcore_barrier()
        pltpu.sync_copy(acc_sh.at[pl.ds(nbase, _NPT), :],
                        out_hbm.at[pl.ds(rowoff + nbase, _NPT), :])
        if sweep == 0:
            plsc.subcore_barrier()


_Z32 = None


def _gat_edges(sd, s, d, xl2):
    src, dst = sd[0], sd[1]
    w, dpart = _edge_pass1(src, dst, s, d)
    dinv = 1.0 / (dpart[0] + dpart[1] + 1e-16)
    zero16 = jnp.zeros((_NPT, 16), F32)
    out = _edge_pass2(src, dst, w, zero16, xl2.reshape(4 * NP, 16))
    return out.reshape(4, NP, 16), dinv


def kernel(x, edge_index, batch, params):
    src = jnp.concatenate([edge_index[0],
                           jnp.zeros((EP - E,), jnp.int32)])
    dst = jnp.concatenate([edge_index[1],
                           jnp.full((EP - E,), N, jnp.int32)])
    sd = jnp.stack([src, dst])
    batch_pad = jnp.concatenate([batch, jnp.full((NP - N,), G, jnp.int32)])
    xp = jnp.pad(x, ((0, NP - N), (0, 0)))

    xc = _in_proj(xp, params['lin_in_W'].T, params['lin_in_b'])
    xloc = jnp.zeros((NP, D), F32)
    for i in range(L):
        p = params['layers'][i]
        xl2, s, d, hh = _pre(xc, p['gat_W'].T, p['att_src'][0, 0],
                             p['att_dst'][0, 0], p['h_W'].T, p['h_b'])
        conv, dinv = _gat_edges(sd, s, d, xl2)
        xc, xloc = _post(conv, dinv, xc, hh, xloc, p['lin_W'].T, p['lin_b'],
                         p['gat_b'], p['ln_w'], p['ln_b'],
                         params['betas'][i])
    return _pool(batch_pad, xloc, params['pred_W'].T, params['pred_b'])


# speedup vs baseline: 27.2711x; 1.2524x over previous
import functools

import jax
import jax.numpy as jnp
from jax import lax
from jax.experimental import pallas as pl
from jax.experimental.pallas import tpu as pltpu
from jax.experimental.pallas import tpu_sc as plsc

N = 50000
E = 800000
IN = 128
D = 64
G = 64
L = 3

NP = 51200
EP = 819200
BR = 1024
GRID = NP // BR

F32 = jnp.float32


def _in_proj_body(x_ref, wt_ref, b_ref, o_ref):
    o_ref[...] = jnp.dot(x_ref[...], wt_ref[...],
                         preferred_element_type=F32) + b_ref[...]


def _in_proj(x, wt, b):
    return pl.pallas_call(
        _in_proj_body,
        grid=(GRID,),
        in_specs=[
            pl.BlockSpec((BR, IN), lambda i: (i, 0)),
            pl.BlockSpec((IN, D), lambda i: (0, 0)),
            pl.BlockSpec((D,), lambda i: (0,)),
        ],
        out_specs=pl.BlockSpec((BR, D), lambda i: (i, 0)),
        out_shape=jax.ShapeDtypeStruct((NP, D), F32),
    )(x, wt, b)


def _pre_body(xc_ref, gwt_ref, asrc_ref, adst_ref, hwt_ref, hb_ref,
              xl_ref, s_ref, d_ref, hh_ref):
    xc = xc_ref[...]
    xl = jnp.dot(xc, gwt_ref[...], preferred_element_type=F32)
    for q in range(4):
        xl_ref[q] = xl[:, 16 * q:16 * q + 16]
    s_ref[...] = jnp.sum(xl * asrc_ref[...][None, :], axis=1)
    d_ref[...] = jnp.sum(xl * adst_ref[...][None, :], axis=1)
    hh_ref[...] = jax.nn.relu(
        jnp.dot(xc, hwt_ref[...], preferred_element_type=F32) + hb_ref[...])


def _pre(xc, gwt, asrc, adst, hwt, hb):
    return pl.pallas_call(
        _pre_body,
        grid=(GRID,),
        in_specs=[
            pl.BlockSpec((BR, D), lambda i: (i, 0)),
            pl.BlockSpec((D, D), lambda i: (0, 0)),
            pl.BlockSpec((D,), lambda i: (0,)),
            pl.BlockSpec((D,), lambda i: (0,)),
            pl.BlockSpec((D, D), lambda i: (0, 0)),
            pl.BlockSpec((D,), lambda i: (0,)),
        ],
        out_specs=[
            pl.BlockSpec((4, BR, 16), lambda i: (0, i, 0)),
            pl.BlockSpec((BR,), lambda i: (i,)),
            pl.BlockSpec((BR,), lambda i: (i,)),
            pl.BlockSpec((BR, D), lambda i: (i, 0)),
        ],
        out_shape=[
            jax.ShapeDtypeStruct((4, NP, 16), F32),
            jax.ShapeDtypeStruct((NP,), F32),
            jax.ShapeDtypeStruct((NP,), F32),
            jax.ShapeDtypeStruct((NP, D), F32),
        ],
    )(xc, gwt, asrc, adst, hwt, hb)


def _post_body(conv_ref, dinv_ref, xc_ref, hh_ref, xloc_ref, lwt_ref, lb_ref,
               gb_ref, lnw_ref, lnb_ref, beta_ref, xcn_ref, xlocn_ref):
    conv = (jnp.concatenate([conv_ref[q] for q in range(4)], axis=1)
            * dinv_ref[...][:, None] + gb_ref[...])
    t = jnp.dot(xc_ref[...], lwt_ref[...],
                preferred_element_type=F32) + lb_ref[...] + conv
    xc2 = jax.nn.relu(t)
    z = hh_ref[...] * xc2
    mu = jnp.mean(z, axis=-1, keepdims=True)
    var = jnp.mean((z - mu) ** 2, axis=-1, keepdims=True)
    ln = (z - mu) / jnp.sqrt(var + 1e-5) * lnw_ref[...] + lnb_ref[...]
    beta = jax.nn.sigmoid(beta_ref[...])[None, :]
    xcn = (1.0 - beta) * ln + beta * xc2
    xcn_ref[...] = xcn
    xlocn_ref[...] = xloc_ref[...] + xcn


def _post(conv, dinv, xc, hh, xloc, lwt, lb, gb, lnw, lnb, beta):
    return pl.pallas_call(
        _post_body,
        grid=(GRID,),
        in_specs=[
            pl.BlockSpec((4, BR, 16), lambda i: (0, i, 0)),
            pl.BlockSpec((BR,), lambda i: (i,)),
            pl.BlockSpec((BR, D), lambda i: (i, 0)),
            pl.BlockSpec((BR, D), lambda i: (i, 0)),
            pl.BlockSpec((BR, D), lambda i: (i, 0)),
            pl.BlockSpec((D, D), lambda i: (0, 0)),
            pl.BlockSpec((D,), lambda i: (0,)),
            pl.BlockSpec((D,), lambda i: (0,)),
            pl.BlockSpec((D,), lambda i: (0,)),
            pl.BlockSpec((D,), lambda i: (0,)),
            pl.BlockSpec((D,), lambda i: (0,)),
        ],
        out_specs=[
            pl.BlockSpec((BR, D), lambda i: (i, 0)),
            pl.BlockSpec((BR, D), lambda i: (i, 0)),
        ],
        out_shape=[
            jax.ShapeDtypeStruct((NP, D), F32),
            jax.ShapeDtypeStruct((NP, D), F32),
        ],
    )(conv, dinv, xc, hh, xloc, lwt, lb, gb, lnw, lnb, beta)


def _pool_body(batch_ref, xl_ref, pwt_ref, pb_ref, o_ref, acc_s, acc_c):
    pid = pl.program_id(0)

    @pl.when(pid == 0)
    def _():
        acc_s[...] = jnp.zeros((G, D), F32)
        acc_c[...] = jnp.zeros((G, D), F32)

    b = batch_ref[...]
    gids = lax.broadcasted_iota(jnp.int32, (G, BR), 0)
    mask = (b[None, :] == gids).astype(F32)
    acc_s[...] += jnp.dot(mask, xl_ref[...], preferred_element_type=F32)
    cnt = jnp.sum(mask, axis=1)
    acc_c[...] += jnp.broadcast_to(cnt[:, None], (G, D))

    @pl.when(pid == GRID - 1)
    def _():
        pooled = acc_s[...] / jnp.maximum(acc_c[...], 1.0)
        o_ref[...] = jnp.dot(pooled, pwt_ref[...],
                             preferred_element_type=F32) + pb_ref[...]


def _pool(batch_pad, xloc, pwt, pb):
    return pl.pallas_call(
        _pool_body,
        grid=(GRID,),
        in_specs=[
            pl.BlockSpec((BR,), lambda i: (i,)),
            pl.BlockSpec((BR, D), lambda i: (i, 0)),
            pl.BlockSpec((D, G), lambda i: (0, 0)),
            pl.BlockSpec((G,), lambda i: (0,)),
        ],
        out_specs=pl.BlockSpec((G, G), lambda i: (0, 0)),
        out_shape=jax.ShapeDtypeStruct((G, G), F32),
        scratch_shapes=[pltpu.VMEM((G, D), F32), pltpu.VMEM((G, D), F32)],
    )(batch_pad, xloc, pwt, pb)


_NC = 2
_NS = 16
_CH = 128
_NPT = NP // _NS
_P1_PER = EP // (_NC * _NS)
_P1_CHUNKS = _P1_PER // _CH
_P2_PER = EP // _NS
_P2_CHUNKS = _P2_PER // _CH

_SCMESH = plsc.VectorSubcoreMesh(core_axis_name="c", subcore_axis_name="s",
                                 num_cores=_NC, num_subcores=_NS)
_K = 4


@functools.partial(
    pl.kernel,
    out_type=[jax.ShapeDtypeStruct((EP,), F32),
              jax.ShapeDtypeStruct((_NC, NP), F32)],
    mesh=_SCMESH,
    compiler_params=pltpu.CompilerParams(use_tc_tiling_on_sc=False),
    scratch_types=[
        pltpu.VMEM((2, _K, _CH), jnp.int32),
        pltpu.VMEM((2, _K, _CH), jnp.int32),
        pltpu.VMEM((2, _K, _CH), F32),
        pltpu.VMEM((2, _K, _CH), F32),
        pltpu.VMEM((2, _K, _CH), F32),
        pltpu.VMEM((_NPT,), F32),
        pltpu.VMEM_SHARED((NP,), F32),
        pltpu.SemaphoreType.DMA,
        pltpu.SemaphoreType.DMA,
        pltpu.SemaphoreType.DMA,
        pltpu.SemaphoreType.DMA,
        pltpu.SemaphoreType.DMA,
        pltpu.SemaphoreType.DMA,
        pltpu.SemaphoreType.DMA,
        pltpu.SemaphoreType.DMA,
        pltpu.SemaphoreType.DMA,
        pltpu.SemaphoreType.DMA,
    ])
def _edge_pass1(src_hbm, dst_hbm, s_hbm, d_hbm, w_hbm, dpart_hbm,
                srcb, dstb, sv, dv, wv, bounce, den_sh,
                semA0, semA1, sg00, sg01, sg02, sg03, sg10, sg11, sg12, sg13):
    cid = lax.axis_index("c")
    sid = lax.axis_index("s")
    wid = cid * _NS + sid
    nbase = sid * _NPT
    semA = [semA0, semA1]
    semG = [[sg00, sg01, sg02, sg03], [sg10, sg11, sg12, sg13]]

    zero = jnp.zeros((16,), F32)

    def zbody(i, _):
        bounce[pl.ds(i * 16, 16)] = zero
        return 0

    lax.fori_loop(0, _NPT // 16, zbody, 0)
    pltpu.sync_copy(bounce, den_sh.at[pl.ds(nbase, _NPT)])
    plsc.subcore_barrier()

    ebase0 = wid * _P1_PER
    n = _P1_CHUNKS // _K

    def issue_loads(k, s):
        gb = ebase0 + k * (_K * _CH)
        for j in range(_K):
            eb = gb + j * _CH
            pltpu.async_copy(src_hbm.at[pl.ds(eb, _CH)], srcb.at[s, j],
                             semA[s])
            pltpu.async_copy(dst_hbm.at[pl.ds(eb, _CH)], dstb.at[s, j],
                             semA[s])

    def drain_loads(k, s):
        gb = ebase0 + k * (_K * _CH)
        for j in range(_K):
            eb = gb + j * _CH
            pltpu.make_async_copy(src_hbm.at[pl.ds(eb, _CH)], srcb.at[s, j],
                                  semA[s]).wait()
            pltpu.make_async_copy(dst_hbm.at[pl.ds(eb, _CH)], dstb.at[s, j],
                                  semA[s]).wait()

    def issue_gathers(s):
        for j in range(_K):
            pltpu.async_copy(s_hbm.at[srcb.at[s, j]], sv.at[s, j], semG[s][j])
            pltpu.async_copy(d_hbm.at[dstb.at[s, j]], dv.at[s, j], semG[s][j])

    def process(k, s):
        gb = ebase0 + k * (_K * _CH)
        for j in range(_K):
            pltpu.make_async_copy(s_hbm.at[srcb.at[s, j]], sv.at[s, j],
                                  semG[s][j]).wait()
            pltpu.make_async_copy(d_hbm.at[dstb.at[s, j]], dv.at[s, j],
                                  semG[s][j]).wait()
            for g in range(_CH // 16):
                a = sv[s, j, pl.ds(g * 16, 16)] + dv[s, j, pl.ds(g * 16, 16)]
                a = jnp.maximum(a, 0.0) + 0.2 * jnp.minimum(a, 0.0)
                wv[s, j, pl.ds(g * 16, 16)] = jnp.exp(a)
            pltpu.sync_copy(wv.at[s, j], w_hbm.at[pl.ds(gb + j * _CH, _CH)])
            pltpu.sync_copy(wv.at[s, j], den_sh.at[dstb.at[s, j]], add=True)

    issue_loads(0, 0)
    drain_loads(0, 0)
    issue_gathers(0)
    issue_loads(1, 1)

    def body(t, _):
        for h in range(2):
            k = 2 * t + h

            @pl.when(k + 1 < n)
            def _():
                drain_loads(k + 1, 1 - h)
                issue_gathers(1 - h)

            process(k, h)

            @pl.when(k + 2 < n)
            def _():
                issue_loads(k + 2, h)
        return 0

    lax.fori_loop(0, n // 2, body, 0)
    plsc.subcore_barrier()
    pltpu.sync_copy(den_sh.at[pl.ds(nbase, _NPT)], bounce)
    pltpu.sync_copy(bounce, dpart_hbm.at[cid].at[pl.ds(nbase, _NPT)])


@functools.partial(
    pl.kernel,
    out_type=jax.ShapeDtypeStruct((4 * NP, 16), F32),
    mesh=_SCMESH,
    compiler_params=pltpu.CompilerParams(use_tc_tiling_on_sc=False),
    scratch_types=[
        pltpu.VMEM((2, _K, _CH), jnp.int32),
        pltpu.VMEM((2, _K, _CH), jnp.int32),
        pltpu.VMEM((2, _K, _CH), jnp.int32),
        pltpu.VMEM((2, _K, _CH), F32),
        pltpu.VMEM((2, _K, _CH, 16), F32),
        pltpu.VMEM((2, _K, _CH, 16), F32),
        pltpu.VMEM_SHARED((NP, 16), F32),
        pltpu.SemaphoreType.DMA,
        pltpu.SemaphoreType.DMA,
        pltpu.SemaphoreType.DMA,
        pltpu.SemaphoreType.DMA,
        pltpu.SemaphoreType.DMA,
        pltpu.SemaphoreType.DMA,
        pltpu.SemaphoreType.DMA,
        pltpu.SemaphoreType.DMA,
        pltpu.SemaphoreType.DMA,
        pltpu.SemaphoreType.DMA,
    ])
def _edge_pass2(src_hbm, dst_hbm, w_hbm, zero_hbm, xl_hbm, out_hbm,
                srcb, dstb, srcb2, wv, rows, msg, acc_sh,
                semA0, semA1, sg00, sg01, sg02, sg03, sg10, sg11, sg12, sg13):
    cid = lax.axis_index("c")
    sid = lax.axis_index("s")
    nbase = sid * _NPT
    semA = [semA0, semA1]
    semG = [[sg00, sg01, sg02, sg03], [sg10, sg11, sg12, sg13]]
    ebase0 = sid * _P2_PER
    n = _P2_CHUNKS // _K

    def issue_loads(k, s):
        gb = ebase0 + k * (_K * _CH)
        for j in range(_K):
            eb = gb + j * _CH
            pltpu.async_copy(src_hbm.at[pl.ds(eb, _CH)], srcb.at[s, j],
                             semA[s])
            pltpu.async_copy(dst_hbm.at[pl.ds(eb, _CH)], dstb.at[s, j],
                             semA[s])
            pltpu.async_copy(w_hbm.at[pl.ds(eb, _CH)], wv.at[s, j], semA[s])

    def drain_loads(k, s):
        gb = ebase0 + k * (_K * _CH)
        for j in range(_K):
            eb = gb + j * _CH
            pltpu.make_async_copy(src_hbm.at[pl.ds(eb, _CH)], srcb.at[s, j],
                                  semA[s]).wait()
            pltpu.make_async_copy(dst_hbm.at[pl.ds(eb, _CH)], dstb.at[s, j],
                                  semA[s]).wait()
            pltpu.make_async_copy(w_hbm.at[pl.ds(eb, _CH)], wv.at[s, j],
                                  semA[s]).wait()

    def issue_gathers(s, rowoff):
        for j in range(_K):
            for g in range(_CH // 16):
                srcb2[s, j, pl.ds(g * 16, 16)] = (
                    srcb[s, j, pl.ds(g * 16, 16)] + rowoff)
            pltpu.async_copy(xl_hbm.at[srcb2.at[s, j]], rows.at[s, j],
                             semG[s][j])

    def process(s):
        for j in range(_K):
            pltpu.make_async_copy(xl_hbm.at[srcb2.at[s, j]], rows.at[s, j],
                                  semG[s][j]).wait()
            for g in range(_CH // 16):
                attg = wv[s, j, pl.ds(g * 16, 16)]
                for t in range(16):
                    e = g * 16 + t
                    msg[s, j, e, pl.ds(0, 16)] = (
                        rows[s, j, e, pl.ds(0, 16)] * attg[t])
            pltpu.sync_copy(msg.at[s, j], acc_sh.at[dstb.at[s, j]], add=True)

    for sweep in range(2):
        rowoff = (2 * sweep + cid) * NP

        pltpu.sync_copy(zero_hbm, acc_sh.at[pl.ds(nbase, _NPT), :])
        plsc.subcore_barrier()

        issue_loads(0, 0)
        drain_loads(0, 0)
        issue_gathers(0, rowoff)
        issue_loads(1, 1)

        def body(t, _):
            for h in range(2):
                k = 2 * t + h

                @pl.when(k + 1 < n)
                def _():
                    drain_loads(k + 1, 1 - h)
                    issue_gathers(1 - h, rowoff)

                process(h)

                @pl.when(k + 2 < n)
                def _():
                    issue_loads(k + 2, h)
            return 0

        lax.fori_loop(0, n // 2, body, 0)
        plsc.subcore_barrier()
        pltpu.sync_copy(acc_sh.at[pl.ds(nbase, _NPT), :],
                        out_hbm.at[pl.ds(rowoff + nbase, _NPT), :])
        if sweep == 0:
            plsc.subcore_barrier()


_Z32 = None


def _gat_edges(src, dst, s, d, xl2):
    w, dpart = _edge_pass1(src, dst, s, d)
    dinv = 1.0 / (dpart[0] + dpart[1] + 1e-16)
    zero16 = jnp.zeros((_NPT, 16), F32)
    out = _edge_pass2(src, dst, w, zero16, xl2.reshape(4 * NP, 16))
    return out.reshape(4, NP, 16), dinv


def kernel(x, edge_index, batch, params):
    src = jnp.concatenate([edge_index[0],
                           jnp.zeros((EP - E,), jnp.int32)])
    dst = jnp.concatenate([edge_index[1],
                           jnp.full((EP - E,), N, jnp.int32)])
    batch_pad = jnp.concatenate([batch, jnp.full((NP - N,), G, jnp.int32)])
    xp = jnp.pad(x, ((0, NP - N), (0, 0)))

    xc = _in_proj(xp, params['lin_in_W'].T, params['lin_in_b'])
    xloc = jnp.zeros((NP, D), F32)
    for i in range(L):
        p = params['layers'][i]
        xl2, s, d, hh = _pre(xc, p['gat_W'].T, p['att_src'][0, 0],
                             p['att_dst'][0, 0], p['h_W'].T, p['h_b'])
        conv, dinv = _gat_edges(src, dst, s, d, xl2)
        xc, xloc = _post(conv, dinv, xc, hh, xloc, p['lin_W'].T, p['lin_b'],
                         p['gat_b'], p['ln_w'], p['ln_b'],
                         params['betas'][i])
    return _pool(batch_pad, xloc, params['pred_W'].T, params['pred_b'])


# speedup vs baseline: 27.2827x; 1.0004x over previous
import functools

import jax
import jax.numpy as jnp
from jax import lax
from jax.experimental import pallas as pl
from jax.experimental.pallas import tpu as pltpu
from jax.experimental.pallas import tpu_sc as plsc

N = 50000
E = 800000
IN = 128
D = 64
G = 64
L = 3

NP = 51200
EP = 819200
BR = 1024
GRID = NP // BR

F32 = jnp.float32


def _in_proj_body(x_ref, wt_ref, b_ref, o_ref):
    o_ref[...] = jnp.dot(x_ref[...], wt_ref[...],
                         preferred_element_type=F32) + b_ref[...]


def _in_proj(x, wt, b):
    return pl.pallas_call(
        _in_proj_body,
        grid=(GRID,),
        in_specs=[
            pl.BlockSpec((BR, IN), lambda i: (i, 0)),
            pl.BlockSpec((IN, D), lambda i: (0, 0)),
            pl.BlockSpec((D,), lambda i: (0,)),
        ],
        out_specs=pl.BlockSpec((BR, D), lambda i: (i, 0)),
        out_shape=jax.ShapeDtypeStruct((NP, D), F32),
    )(x, wt, b)


def _pre_body(xc_ref, gwt_ref, asrc_ref, adst_ref, hwt_ref, hb_ref,
              xl_ref, s_ref, d_ref, hh_ref):
    xc = xc_ref[...]
    xl = jnp.dot(xc, gwt_ref[...], preferred_element_type=F32)
    for q in range(4):
        xl_ref[q] = xl[:, 16 * q:16 * q + 16]
    s_ref[...] = jnp.sum(xl * asrc_ref[...][None, :], axis=1)
    d_ref[...] = jnp.sum(xl * adst_ref[...][None, :], axis=1)
    hh_ref[...] = jax.nn.relu(
        jnp.dot(xc, hwt_ref[...], preferred_element_type=F32) + hb_ref[...])


def _pre(xc, gwt, asrc, adst, hwt, hb):
    return pl.pallas_call(
        _pre_body,
        grid=(GRID,),
        in_specs=[
            pl.BlockSpec((BR, D), lambda i: (i, 0)),
            pl.BlockSpec((D, D), lambda i: (0, 0)),
            pl.BlockSpec((D,), lambda i: (0,)),
            pl.BlockSpec((D,), lambda i: (0,)),
            pl.BlockSpec((D, D), lambda i: (0, 0)),
            pl.BlockSpec((D,), lambda i: (0,)),
        ],
        out_specs=[
            pl.BlockSpec((4, BR, 16), lambda i: (0, i, 0)),
            pl.BlockSpec((BR,), lambda i: (i,)),
            pl.BlockSpec((BR,), lambda i: (i,)),
            pl.BlockSpec((BR, D), lambda i: (i, 0)),
        ],
        out_shape=[
            jax.ShapeDtypeStruct((4, NP, 16), F32),
            jax.ShapeDtypeStruct((NP,), F32),
            jax.ShapeDtypeStruct((NP,), F32),
            jax.ShapeDtypeStruct((NP, D), F32),
        ],
    )(xc, gwt, asrc, adst, hwt, hb)


def _post_body(conv_ref, dinv_ref, xc_ref, hh_ref, xloc_ref, lwt_ref, lb_ref,
               gb_ref, lnw_ref, lnb_ref, beta_ref, xcn_ref, xlocn_ref):
    conv = (jnp.concatenate([conv_ref[q] for q in range(4)], axis=1)
            * dinv_ref[...][:, None] + gb_ref[...])
    t = jnp.dot(xc_ref[...], lwt_ref[...],
                preferred_element_type=F32) + lb_ref[...] + conv
    xc2 = jax.nn.relu(t)
    z = hh_ref[...] * xc2
    mu = jnp.mean(z, axis=-1, keepdims=True)
    var = jnp.mean((z - mu) ** 2, axis=-1, keepdims=True)
    ln = (z - mu) / jnp.sqrt(var + 1e-5) * lnw_ref[...] + lnb_ref[...]
    beta = jax.nn.sigmoid(beta_ref[...])[None, :]
    xcn = (1.0 - beta) * ln + beta * xc2
    xcn_ref[...] = xcn
    xlocn_ref[...] = xloc_ref[...] + xcn


def _post(conv, dinv, xc, hh, xloc, lwt, lb, gb, lnw, lnb, beta):
    return pl.pallas_call(
        _post_body,
        grid=(GRID,),
        in_specs=[
            pl.BlockSpec((4, BR, 16), lambda i: (0, i, 0)),
            pl.BlockSpec((BR,), lambda i: (i,)),
            pl.BlockSpec((BR, D), lambda i: (i, 0)),
            pl.BlockSpec((BR, D), lambda i: (i, 0)),
            pl.BlockSpec((BR, D), lambda i: (i, 0)),
            pl.BlockSpec((D, D), lambda i: (0, 0)),
            pl.BlockSpec((D,), lambda i: (0,)),
            pl.BlockSpec((D,), lambda i: (0,)),
            pl.BlockSpec((D,), lambda i: (0,)),
            pl.BlockSpec((D,), lambda i: (0,)),
            pl.BlockSpec((D,), lambda i: (0,)),
        ],
        out_specs=[
            pl.BlockSpec((BR, D), lambda i: (i, 0)),
            pl.BlockSpec((BR, D), lambda i: (i, 0)),
        ],
        out_shape=[
            jax.ShapeDtypeStruct((NP, D), F32),
            jax.ShapeDtypeStruct((NP, D), F32),
        ],
    )(conv, dinv, xc, hh, xloc, lwt, lb, gb, lnw, lnb, beta)


def _pool_body(batch_ref, xl_ref, pwt_ref, pb_ref, o_ref, acc_s, acc_c):
    pid = pl.program_id(0)

    @pl.when(pid == 0)
    def _():
        acc_s[...] = jnp.zeros((G, D), F32)
        acc_c[...] = jnp.zeros((G, D), F32)

    b = batch_ref[...]
    gids = lax.broadcasted_iota(jnp.int32, (G, BR), 0)
    mask = (b[None, :] == gids).astype(F32)
    acc_s[...] += jnp.dot(mask, xl_ref[...], preferred_element_type=F32)
    cnt = jnp.sum(mask, axis=1)
    acc_c[...] += jnp.broadcast_to(cnt[:, None], (G, D))

    @pl.when(pid == GRID - 1)
    def _():
        pooled = acc_s[...] / jnp.maximum(acc_c[...], 1.0)
        o_ref[...] = jnp.dot(pooled, pwt_ref[...],
                             preferred_element_type=F32) + pb_ref[...]


def _pool(batch_pad, xloc, pwt, pb):
    return pl.pallas_call(
        _pool_body,
        grid=(GRID,),
        in_specs=[
            pl.BlockSpec((BR,), lambda i: (i,)),
            pl.BlockSpec((BR, D), lambda i: (i, 0)),
            pl.BlockSpec((D, G), lambda i: (0, 0)),
            pl.BlockSpec((G,), lambda i: (0,)),
        ],
        out_specs=pl.BlockSpec((G, G), lambda i: (0, 0)),
        out_shape=jax.ShapeDtypeStruct((G, G), F32),
        scratch_shapes=[pltpu.VMEM((G, D), F32), pltpu.VMEM((G, D), F32)],
    )(batch_pad, xloc, pwt, pb)


_NC = 2
_NS = 16
_CH = 128
_NPT = NP // _NS
_P1_PER = EP // (_NC * _NS)
_P1_CHUNKS = _P1_PER // _CH
_P2_PER = EP // _NS
_P2_CHUNKS = _P2_PER // _CH

_SCMESH = plsc.VectorSubcoreMesh(core_axis_name="c", subcore_axis_name="s",
                                 num_cores=_NC, num_subcores=_NS)
_K = 4


@functools.partial(
    pl.kernel,
    out_type=[jax.ShapeDtypeStruct((EP,), F32),
              jax.ShapeDtypeStruct((_NC, NP), F32)],
    mesh=_SCMESH,
    compiler_params=pltpu.CompilerParams(use_tc_tiling_on_sc=False),
    scratch_types=[
        pltpu.VMEM((2, _K, _CH), jnp.int32),
        pltpu.VMEM((2, _K, _CH), jnp.int32),
        pltpu.VMEM((2, _K, _CH), F32),
        pltpu.VMEM((2, _K, _CH), F32),
        pltpu.VMEM((2, _K, _CH), F32),
        pltpu.VMEM((_NPT,), F32),
        pltpu.VMEM_SHARED((NP,), F32),
        pltpu.SemaphoreType.DMA,
        pltpu.SemaphoreType.DMA,
        pltpu.SemaphoreType.DMA,
        pltpu.SemaphoreType.DMA,
        pltpu.SemaphoreType.DMA,
        pltpu.SemaphoreType.DMA,
        pltpu.SemaphoreType.DMA,
        pltpu.SemaphoreType.DMA,
        pltpu.SemaphoreType.DMA,
        pltpu.SemaphoreType.DMA,
        pltpu.SemaphoreType.DMA,
        pltpu.SemaphoreType.DMA,
    ])
def _edge_pass1(src_hbm, dst_hbm, s_hbm, d_hbm, w_hbm, dpart_hbm,
                srcb, dstb, sv, dv, wv, bounce, den_sh,
                semA0, semA1, sg00, sg01, sg02, sg03, sg10, sg11, sg12, sg13,
                semW0, semW1):
    cid = lax.axis_index("c")
    sid = lax.axis_index("s")
    wid = cid * _NS + sid
    nbase = sid * _NPT
    semA = [semA0, semA1]
    semG = [[sg00, sg01, sg02, sg03], [sg10, sg11, sg12, sg13]]
    semW = [semW0, semW1]

    zero = jnp.zeros((16,), F32)

    def zbody(i, _):
        bounce[pl.ds(i * 16, 16)] = zero
        return 0

    lax.fori_loop(0, _NPT // 16, zbody, 0)
    pltpu.sync_copy(bounce, den_sh.at[pl.ds(nbase, _NPT)])
    plsc.subcore_barrier()

    ebase0 = wid * _P1_PER
    n = _P1_CHUNKS // _K

    def issue_loads(k, s):
        gb = ebase0 + k * (_K * _CH)
        for j in range(_K):
            eb = gb + j * _CH
            pltpu.async_copy(src_hbm.at[pl.ds(eb, _CH)], srcb.at[s, j],
                             semA[s])
            pltpu.async_copy(dst_hbm.at[pl.ds(eb, _CH)], dstb.at[s, j],
                             semA[s])

    def drain_loads(k, s):
        gb = ebase0 + k * (_K * _CH)
        for j in range(_K):
            eb = gb + j * _CH
            pltpu.make_async_copy(src_hbm.at[pl.ds(eb, _CH)], srcb.at[s, j],
                                  semA[s]).wait()
            pltpu.make_async_copy(dst_hbm.at[pl.ds(eb, _CH)], dstb.at[s, j],
                                  semA[s]).wait()

    def issue_gathers(s):
        for j in range(_K):
            pltpu.async_copy(s_hbm.at[srcb.at[s, j]], sv.at[s, j], semG[s][j])
            pltpu.async_copy(d_hbm.at[dstb.at[s, j]], dv.at[s, j], semG[s][j])

    def drain_writes(k, s):
        gb = ebase0 + k * (_K * _CH)
        for j in range(_K):
            pltpu.make_async_copy(wv.at[s, j],
                                  w_hbm.at[pl.ds(gb + j * _CH, _CH)],
                                  semW[s]).wait()

    def process(k, s):
        gb = ebase0 + k * (_K * _CH)

        @pl.when(k >= 2)
        def _():
            drain_writes(k - 2, s)

        for j in range(_K):
            pltpu.make_async_copy(s_hbm.at[srcb.at[s, j]], sv.at[s, j],
                                  semG[s][j]).wait()
            pltpu.make_async_copy(d_hbm.at[dstb.at[s, j]], dv.at[s, j],
                                  semG[s][j]).wait()
            for g in range(_CH // 16):
                a = sv[s, j, pl.ds(g * 16, 16)] + dv[s, j, pl.ds(g * 16, 16)]
                a = jnp.maximum(a, 0.0) + 0.2 * jnp.minimum(a, 0.0)
                wv[s, j, pl.ds(g * 16, 16)] = jnp.exp(a)
            pltpu.async_copy(wv.at[s, j], w_hbm.at[pl.ds(gb + j * _CH, _CH)],
                             semW[s])
            pltpu.sync_copy(wv.at[s, j], den_sh.at[dstb.at[s, j]], add=True)

    issue_loads(0, 0)
    drain_loads(0, 0)
    issue_gathers(0)
    issue_loads(1, 1)

    def body(t, _):
        for h in range(2):
            k = 2 * t + h

            @pl.when(k + 1 < n)
            def _():
                drain_loads(k + 1, 1 - h)
                issue_gathers(1 - h)

            process(k, h)

            @pl.when(k + 2 < n)
            def _():
                issue_loads(k + 2, h)
        return 0

    lax.fori_loop(0, n // 2, body, 0)
    drain_writes(n - 2, 0)
    drain_writes(n - 1, 1)
    plsc.subcore_barrier()
    pltpu.sync_copy(den_sh.at[pl.ds(nbase, _NPT)], bounce)
    pltpu.sync_copy(bounce, dpart_hbm.at[cid].at[pl.ds(nbase, _NPT)])


@functools.partial(
    pl.kernel,
    out_type=jax.ShapeDtypeStruct((4 * NP, 16), F32),
    mesh=_SCMESH,
    compiler_params=pltpu.CompilerParams(use_tc_tiling_on_sc=False),
    scratch_types=[
        pltpu.VMEM((2, _K, _CH), jnp.int32),
        pltpu.VMEM((2, _K, _CH), jnp.int32),
        pltpu.VMEM((2, _K, _CH), jnp.int32),
        pltpu.VMEM((2, _K, _CH), F32),
        pltpu.VMEM((2, _K, _CH, 16), F32),
        pltpu.VMEM((2, _K, _CH, 16), F32),
        pltpu.VMEM_SHARED((NP, 16), F32),
        pltpu.SemaphoreType.DMA,
        pltpu.SemaphoreType.DMA,
        pltpu.SemaphoreType.DMA,
        pltpu.SemaphoreType.DMA,
        pltpu.SemaphoreType.DMA,
        pltpu.SemaphoreType.DMA,
        pltpu.SemaphoreType.DMA,
        pltpu.SemaphoreType.DMA,
        pltpu.SemaphoreType.DMA,
        pltpu.SemaphoreType.DMA,
    ])
def _edge_pass2(src_hbm, dst_hbm, w_hbm, zero_hbm, xl_hbm, out_hbm,
                srcb, dstb, srcb2, wv, rows, msg, acc_sh,
                semA0, semA1, sg00, sg01, sg02, sg03, sg10, sg11, sg12, sg13):
    cid = lax.axis_index("c")
    sid = lax.axis_index("s")
    nbase = sid * _NPT
    semA = [semA0, semA1]
    semG = [[sg00, sg01, sg02, sg03], [sg10, sg11, sg12, sg13]]
    ebase0 = sid * _P2_PER
    n = _P2_CHUNKS // _K

    def issue_loads(k, s):
        gb = ebase0 + k * (_K * _CH)
        for j in range(_K):
            eb = gb + j * _CH
            pltpu.async_copy(src_hbm.at[pl.ds(eb, _CH)], srcb.at[s, j],
                             semA[s])
            pltpu.async_copy(dst_hbm.at[pl.ds(eb, _CH)], dstb.at[s, j],
                             semA[s])
            pltpu.async_copy(w_hbm.at[pl.ds(eb, _CH)], wv.at[s, j], semA[s])

    def drain_loads(k, s):
        gb = ebase0 + k * (_K * _CH)
        for j in range(_K):
            eb = gb + j * _CH
            pltpu.make_async_copy(src_hbm.at[pl.ds(eb, _CH)], srcb.at[s, j],
                                  semA[s]).wait()
            pltpu.make_async_copy(dst_hbm.at[pl.ds(eb, _CH)], dstb.at[s, j],
                                  semA[s]).wait()
            pltpu.make_async_copy(w_hbm.at[pl.ds(eb, _CH)], wv.at[s, j],
                                  semA[s]).wait()

    def issue_gathers(s, rowoff):
        for j in range(_K):
            for g in range(_CH // 16):
                srcb2[s, j, pl.ds(g * 16, 16)] = (
                    srcb[s, j, pl.ds(g * 16, 16)] + rowoff)
            pltpu.async_copy(xl_hbm.at[srcb2.at[s, j]], rows.at[s, j],
                             semG[s][j])

    def process(k, s):
        for j in range(_K):
            pltpu.make_async_copy(xl_hbm.at[srcb2.at[s, j]], rows.at[s, j],
                                  semG[s][j]).wait()
            for g in range(_CH // 16):
                attg = wv[s, j, pl.ds(g * 16, 16)]
                for t in range(16):
                    e = g * 16 + t
                    msg[s, j, e, pl.ds(0, 16)] = (
                        rows[s, j, e, pl.ds(0, 16)] * attg[t])
            pltpu.sync_copy(msg.at[s, j], acc_sh.at[dstb.at[s, j]], add=True)

    for sweep in range(2):
        rowoff = (2 * sweep + cid) * NP

        pltpu.sync_copy(zero_hbm, acc_sh.at[pl.ds(nbase, _NPT), :])
        plsc.subcore_barrier()

        issue_loads(0, 0)
        drain_loads(0, 0)
        issue_gathers(0, rowoff)
        issue_loads(1, 1)

        def body(t, _):
            for h in range(2):
                k = 2 * t + h

                @pl.when(k + 1 < n)
                def _():
                    drain_loads(k + 1, 1 - h)
                    issue_gathers(1 - h, rowoff)

                process(k, h)

                @pl.when(k + 2 < n)
                def _():
                    issue_loads(k + 2, h)
            return 0

        lax.fori_loop(0, n // 2, body, 0)
        plsc.subcore_barrier()
        pltpu.sync_copy(acc_sh.at[pl.ds(nbase, _NPT), :],
                        out_hbm.at[pl.ds(rowoff + nbase, _NPT), :])
        if sweep == 0:
            plsc.subcore_barrier()


_Z32 = None


def _gat_edges(src, dst, s, d, xl2):
    w, dpart = _edge_pass1(src, dst, s, d)
    dinv = 1.0 / (dpart[0] + dpart[1] + 1e-16)
    zero16 = jnp.zeros((_NPT, 16), F32)
    out = _edge_pass2(src, dst, w, zero16, xl2.reshape(4 * NP, 16))
    return out.reshape(4, NP, 16), dinv


def kernel(x, edge_index, batch, params):
    src = jnp.concatenate([edge_index[0],
                           jnp.zeros((EP - E,), jnp.int32)])
    dst = jnp.concatenate([edge_index[1],
                           jnp.full((EP - E,), N, jnp.int32)])
    batch_pad = jnp.concatenate([batch, jnp.full((NP - N,), G, jnp.int32)])
    xp = jnp.pad(x, ((0, NP - N), (0, 0)))

    xc = _in_proj(xp, params['lin_in_W'].T, params['lin_in_b'])
    xloc = jnp.zeros((NP, D), F32)
    for i in range(L):
        p = params['layers'][i]
        xl2, s, d, hh = _pre(xc, p['gat_W'].T, p['att_src'][0, 0],
                             p['att_dst'][0, 0], p['h_W'].T, p['h_b'])
        conv, dinv = _gat_edges(src, dst, s, d, xl2)
        xc, xloc = _post(conv, dinv, xc, hh, xloc, p['lin_W'].T, p['lin_b'],
                         p['gat_b'], p['ln_w'], p['ln_b'],
                         params['betas'][i])
    return _pool(batch_pad, xloc, params['pred_W'].T, params['pred_b'])


# speedup vs baseline: 28.2211x; 1.0344x over previous
import functools

import jax
import jax.numpy as jnp
from jax import lax
from jax.experimental import pallas as pl
from jax.experimental.pallas import tpu as pltpu
from jax.experimental.pallas import tpu_sc as plsc

N = 50000
E = 800000
IN = 128
D = 64
G = 64
L = 3

NP = 51200
EP = 819200
BR = 1024
GRID = NP // BR

F32 = jnp.float32


def _pre_part(xc, gwt_ref, asrc_ref, adst_ref, hwt_ref, hb_ref,
              xl_ref, s_ref, d_ref, hh_ref):
    xl = jnp.dot(xc, gwt_ref[...], preferred_element_type=F32)
    for q in range(4):
        xl_ref[q] = xl[:, 16 * q:16 * q + 16]
    s_ref[...] = jnp.sum(xl * asrc_ref[...][None, :], axis=1)
    d_ref[...] = jnp.sum(xl * adst_ref[...][None, :], axis=1)
    hh_ref[...] = jax.nn.relu(
        jnp.dot(xc, hwt_ref[...], preferred_element_type=F32) + hb_ref[...])


def _post_part(conv_ref, dinv_ref, xc_ref, hh_ref, xloc_ref, lwt_ref, lb_ref,
               gb_ref, lnw_ref, lnb_ref, beta_ref):
    conv = (jnp.concatenate([conv_ref[q] for q in range(4)], axis=1)
            * dinv_ref[...][:, None] + gb_ref[...])
    t = jnp.dot(xc_ref[...], lwt_ref[...],
                preferred_element_type=F32) + lb_ref[...] + conv
    xc2 = jax.nn.relu(t)
    z = hh_ref[...] * xc2
    mu = jnp.mean(z, axis=-1, keepdims=True)
    var = jnp.mean((z - mu) ** 2, axis=-1, keepdims=True)
    ln = (z - mu) / jnp.sqrt(var + 1e-5) * lnw_ref[...] + lnb_ref[...]
    beta = jax.nn.sigmoid(beta_ref[...])[None, :]
    xcn = (1.0 - beta) * ln + beta * xc2
    return xcn, xloc_ref[...] + xcn


_VSPEC = pl.BlockSpec((D,), lambda i: (0,))
_MSPEC = pl.BlockSpec((BR, D), lambda i: (i, 0))
_WSPEC = pl.BlockSpec((D, D), lambda i: (0, 0))
_SSPEC = pl.BlockSpec((BR,), lambda i: (i,))
_XLSPEC = pl.BlockSpec((4, BR, 16), lambda i: (0, i, 0))


def _in_pre_body(x_ref, wt_ref, b_ref, gwt_ref, asrc_ref, adst_ref, hwt_ref,
                 hb_ref, xc_ref, xl_ref, s_ref, d_ref, hh_ref):
    xc = jnp.dot(x_ref[...], wt_ref[...],
                 preferred_element_type=F32) + b_ref[...]
    xc_ref[...] = xc
    _pre_part(xc, gwt_ref, asrc_ref, adst_ref, hwt_ref, hb_ref,
              xl_ref, s_ref, d_ref, hh_ref)


def _in_pre(x, wt, b, gwt, asrc, adst, hwt, hb):
    return pl.pallas_call(
        _in_pre_body,
        grid=(GRID,),
        in_specs=[
            pl.BlockSpec((BR, IN), lambda i: (i, 0)),
            pl.BlockSpec((IN, D), lambda i: (0, 0)),
            _VSPEC, _WSPEC, _VSPEC, _VSPEC, _WSPEC, _VSPEC,
        ],
        out_specs=[_MSPEC, _XLSPEC, _SSPEC, _SSPEC, _MSPEC],
        out_shape=[
            jax.ShapeDtypeStruct((NP, D), F32),
            jax.ShapeDtypeStruct((4, NP, 16), F32),
            jax.ShapeDtypeStruct((NP,), F32),
            jax.ShapeDtypeStruct((NP,), F32),
            jax.ShapeDtypeStruct((NP, D), F32),
        ],
    )(x, wt, b, gwt, asrc, adst, hwt, hb)


def _post_pre_body(conv_ref, dinv_ref, xc_ref, hh_ref, xloc_ref, lwt_ref,
                   lb_ref, gb_ref, lnw_ref, lnb_ref, beta_ref,
                   gwt_ref, asrc_ref, adst_ref, hwt_ref, hb_ref,
                   xcn_ref, xlocn_ref, xl_ref, s_ref, d_ref, hh2_ref):
    xcn, xlocn = _post_part(conv_ref, dinv_ref, xc_ref, hh_ref, xloc_ref,
                            lwt_ref, lb_ref, gb_ref, lnw_ref, lnb_ref,
                            beta_ref)
    xcn_ref[...] = xcn
    xlocn_ref[...] = xlocn
    _pre_part(xcn, gwt_ref, asrc_ref, adst_ref, hwt_ref, hb_ref,
              xl_ref, s_ref, d_ref, hh2_ref)


def _post_pre(conv, dinv, xc, hh, xloc, lwt, lb, gb, lnw, lnb, beta,
              gwt, asrc, adst, hwt, hb):
    return pl.pallas_call(
        _post_pre_body,
        grid=(GRID,),
        in_specs=[
            _XLSPEC, _SSPEC, _MSPEC, _MSPEC, _MSPEC,
            _WSPEC, _VSPEC, _VSPEC, _VSPEC, _VSPEC, _VSPEC,
            _WSPEC, _VSPEC, _VSPEC, _WSPEC, _VSPEC,
        ],
        out_specs=[_MSPEC, _MSPEC, _XLSPEC, _SSPEC, _SSPEC, _MSPEC],
        out_shape=[
            jax.ShapeDtypeStruct((NP, D), F32),
            jax.ShapeDtypeStruct((NP, D), F32),
            jax.ShapeDtypeStruct((4, NP, 16), F32),
            jax.ShapeDtypeStruct((NP,), F32),
            jax.ShapeDtypeStruct((NP,), F32),
            jax.ShapeDtypeStruct((NP, D), F32),
        ],
    )(conv, dinv, xc, hh, xloc, lwt, lb, gb, lnw, lnb, beta,
      gwt, asrc, adst, hwt, hb)


def _post_pool_body(conv_ref, dinv_ref, xc_ref, hh_ref, xloc_ref, lwt_ref,
                    lb_ref, gb_ref, lnw_ref, lnb_ref, beta_ref,
                    batch_ref, pwt_ref, pb_ref, o_ref, acc_s, acc_c):
    pid = pl.program_id(0)

    @pl.when(pid == 0)
    def _():
        acc_s[...] = jnp.zeros((G, D), F32)
        acc_c[...] = jnp.zeros((G, D), F32)

    _, xlocn = _post_part(conv_ref, dinv_ref, xc_ref, hh_ref, xloc_ref,
                          lwt_ref, lb_ref, gb_ref, lnw_ref, lnb_ref,
                          beta_ref)
    b = batch_ref[...]
    gids = lax.broadcasted_iota(jnp.int32, (G, BR), 0)
    mask = (b[None, :] == gids).astype(F32)
    acc_s[...] += jnp.dot(mask, xlocn, preferred_element_type=F32)
    cnt = jnp.sum(mask, axis=1)
    acc_c[...] += jnp.broadcast_to(cnt[:, None], (G, D))

    @pl.when(pid == GRID - 1)
    def _():
        pooled = acc_s[...] / jnp.maximum(acc_c[...], 1.0)
        o_ref[...] = jnp.dot(pooled, pwt_ref[...],
                             preferred_element_type=F32) + pb_ref[...]


def _post_pool(conv, dinv, xc, hh, xloc, lwt, lb, gb, lnw, lnb, beta,
               batch_pad, pwt, pb):
    return pl.pallas_call(
        _post_pool_body,
        grid=(GRID,),
        in_specs=[
            _XLSPEC, _SSPEC, _MSPEC, _MSPEC, _MSPEC,
            _WSPEC, _VSPEC, _VSPEC, _VSPEC, _VSPEC, _VSPEC,
            _SSPEC,
            pl.BlockSpec((D, G), lambda i: (0, 0)),
            pl.BlockSpec((G,), lambda i: (0,)),
        ],
        out_specs=pl.BlockSpec((G, G), lambda i: (0, 0)),
        out_shape=jax.ShapeDtypeStruct((G, G), F32),
        scratch_shapes=[pltpu.VMEM((G, D), F32), pltpu.VMEM((G, D), F32)],
    )(conv, dinv, xc, hh, xloc, lwt, lb, gb, lnw, lnb, beta,
      batch_pad, pwt, pb)


_NC = 2
_NS = 16
_CH = 128
_NPT = NP // _NS
_P1_PER = EP // (_NC * _NS)
_P1_CHUNKS = _P1_PER // _CH
_P2_PER = EP // _NS
_P2_CHUNKS = _P2_PER // _CH

_SCMESH = plsc.VectorSubcoreMesh(core_axis_name="c", subcore_axis_name="s",
                                 num_cores=_NC, num_subcores=_NS)
_K = 4


@functools.partial(
    pl.kernel,
    out_type=[jax.ShapeDtypeStruct((EP,), F32),
              jax.ShapeDtypeStruct((_NC, NP), F32)],
    mesh=_SCMESH,
    compiler_params=pltpu.CompilerParams(use_tc_tiling_on_sc=False),
    scratch_types=[
        pltpu.VMEM((2, _K, _CH), jnp.int32),
        pltpu.VMEM((2, _K, _CH), jnp.int32),
        pltpu.VMEM((2, _K, _CH), F32),
        pltpu.VMEM((2, _K, _CH), F32),
        pltpu.VMEM((2, _K, _CH), F32),
        pltpu.VMEM((_NPT,), F32),
        pltpu.VMEM_SHARED((NP,), F32),
        pltpu.SemaphoreType.DMA,
        pltpu.SemaphoreType.DMA,
        pltpu.SemaphoreType.DMA,
        pltpu.SemaphoreType.DMA,
        pltpu.SemaphoreType.DMA,
        pltpu.SemaphoreType.DMA,
        pltpu.SemaphoreType.DMA,
        pltpu.SemaphoreType.DMA,
        pltpu.SemaphoreType.DMA,
        pltpu.SemaphoreType.DMA,
        pltpu.SemaphoreType.DMA,
        pltpu.SemaphoreType.DMA,
    ])
def _edge_pass1(src_hbm, dst_hbm, s_hbm, d_hbm, w_hbm, dpart_hbm,
                srcb, dstb, sv, dv, wv, bounce, den_sh,
                semA0, semA1, sg00, sg01, sg02, sg03, sg10, sg11, sg12, sg13,
                semW0, semW1):
    cid = lax.axis_index("c")
    sid = lax.axis_index("s")
    wid = cid * _NS + sid
    nbase = sid * _NPT
    semA = [semA0, semA1]
    semG = [[sg00, sg01, sg02, sg03], [sg10, sg11, sg12, sg13]]
    semW = [semW0, semW1]

    zero = jnp.zeros((16,), F32)

    def zbody(i, _):
        bounce[pl.ds(i * 16, 16)] = zero
        return 0

    lax.fori_loop(0, _NPT // 16, zbody, 0)
    pltpu.sync_copy(bounce, den_sh.at[pl.ds(nbase, _NPT)])
    plsc.subcore_barrier()

    ebase0 = wid * _P1_PER
    n = _P1_CHUNKS // _K

    def issue_loads(k, s):
        gb = ebase0 + k * (_K * _CH)
        for j in range(_K):
            eb = gb + j * _CH
            pltpu.async_copy(src_hbm.at[pl.ds(eb, _CH)], srcb.at[s, j],
                             semA[s])
            pltpu.async_copy(dst_hbm.at[pl.ds(eb, _CH)], dstb.at[s, j],
                             semA[s])

    def drain_loads(k, s):
        gb = ebase0 + k * (_K * _CH)
        for j in range(_K):
            eb = gb + j * _CH
            pltpu.make_async_copy(src_hbm.at[pl.ds(eb, _CH)], srcb.at[s, j],
                                  semA[s]).wait()
            pltpu.make_async_copy(dst_hbm.at[pl.ds(eb, _CH)], dstb.at[s, j],
                                  semA[s]).wait()

    def issue_gathers(s):
        for j in range(_K):
            pltpu.async_copy(s_hbm.at[srcb.at[s, j]], sv.at[s, j], semG[s][j])
            pltpu.async_copy(d_hbm.at[dstb.at[s, j]], dv.at[s, j], semG[s][j])

    def drain_writes(k, s):
        gb = ebase0 + k * (_K * _CH)
        for j in range(_K):
            pltpu.make_async_copy(wv.at[s, j],
                                  w_hbm.at[pl.ds(gb + j * _CH, _CH)],
                                  semW[s]).wait()

    def process(k, s):
        gb = ebase0 + k * (_K * _CH)

        @pl.when(k >= 2)
        def _():
            drain_writes(k - 2, s)

        for j in range(_K):
            pltpu.make_async_copy(s_hbm.at[srcb.at[s, j]], sv.at[s, j],
                                  semG[s][j]).wait()
            pltpu.make_async_copy(d_hbm.at[dstb.at[s, j]], dv.at[s, j],
                                  semG[s][j]).wait()
            for g in range(_CH // 16):
                a = sv[s, j, pl.ds(g * 16, 16)] + dv[s, j, pl.ds(g * 16, 16)]
                a = jnp.maximum(a, 0.0) + 0.2 * jnp.minimum(a, 0.0)
                wv[s, j, pl.ds(g * 16, 16)] = jnp.exp(a)
            pltpu.async_copy(wv.at[s, j], w_hbm.at[pl.ds(gb + j * _CH, _CH)],
                             semW[s])
            pltpu.sync_copy(wv.at[s, j], den_sh.at[dstb.at[s, j]], add=True)

    issue_loads(0, 0)
    drain_loads(0, 0)
    issue_gathers(0)
    issue_loads(1, 1)

    def body(t, _):
        for h in range(2):
            k = 2 * t + h

            @pl.when(k + 1 < n)
            def _():
                drain_loads(k + 1, 1 - h)
                issue_gathers(1 - h)

            process(k, h)

            @pl.when(k + 2 < n)
            def _():
                issue_loads(k + 2, h)
        return 0

    lax.fori_loop(0, n // 2, body, 0)
    drain_writes(n - 2, 0)
    drain_writes(n - 1, 1)
    plsc.subcore_barrier()
    pltpu.sync_copy(den_sh.at[pl.ds(nbase, _NPT)], bounce)
    pltpu.sync_copy(bounce, dpart_hbm.at[cid].at[pl.ds(nbase, _NPT)])


@functools.partial(
    pl.kernel,
    out_type=jax.ShapeDtypeStruct((4 * NP, 16), F32),
    mesh=_SCMESH,
    compiler_params=pltpu.CompilerParams(use_tc_tiling_on_sc=False),
    scratch_types=[
        pltpu.VMEM((2, _K, _CH), jnp.int32),
        pltpu.VMEM((2, _K, _CH), jnp.int32),
        pltpu.VMEM((2, _K, _CH), jnp.int32),
        pltpu.VMEM((2, _K, _CH), F32),
        pltpu.VMEM((2, _K, _CH, 16), F32),
        pltpu.VMEM((2, _K, _CH, 16), F32),
        pltpu.VMEM_SHARED((NP, 16), F32),
        pltpu.SemaphoreType.DMA,
        pltpu.SemaphoreType.DMA,
        pltpu.SemaphoreType.DMA,
        pltpu.SemaphoreType.DMA,
        pltpu.SemaphoreType.DMA,
        pltpu.SemaphoreType.DMA,
        pltpu.SemaphoreType.DMA,
        pltpu.SemaphoreType.DMA,
        pltpu.SemaphoreType.DMA,
        pltpu.SemaphoreType.DMA,
    ])
def _edge_pass2(src_hbm, dst_hbm, w_hbm, zero_hbm, xl_hbm, out_hbm,
                srcb, dstb, srcb2, wv, rows, msg, acc_sh,
                semA0, semA1, sg00, sg01, sg02, sg03, sg10, sg11, sg12, sg13):
    cid = lax.axis_index("c")
    sid = lax.axis_index("s")
    nbase = sid * _NPT
    semA = [semA0, semA1]
    semG = [[sg00, sg01, sg02, sg03], [sg10, sg11, sg12, sg13]]
    ebase0 = sid * _P2_PER
    n = _P2_CHUNKS // _K

    def issue_loads(k, s):
        gb = ebase0 + k * (_K * _CH)
        for j in range(_K):
            eb = gb + j * _CH
            pltpu.async_copy(src_hbm.at[pl.ds(eb, _CH)], srcb.at[s, j],
                             semA[s])
            pltpu.async_copy(dst_hbm.at[pl.ds(eb, _CH)], dstb.at[s, j],
                             semA[s])
            pltpu.async_copy(w_hbm.at[pl.ds(eb, _CH)], wv.at[s, j], semA[s])

    def drain_loads(k, s):
        gb = ebase0 + k * (_K * _CH)
        for j in range(_K):
            eb = gb + j * _CH
            pltpu.make_async_copy(src_hbm.at[pl.ds(eb, _CH)], srcb.at[s, j],
                                  semA[s]).wait()
            pltpu.make_async_copy(dst_hbm.at[pl.ds(eb, _CH)], dstb.at[s, j],
                                  semA[s]).wait()
            pltpu.make_async_copy(w_hbm.at[pl.ds(eb, _CH)], wv.at[s, j],
                                  semA[s]).wait()

    def issue_gathers(s, rowoff):
        for j in range(_K):
            for g in range(_CH // 16):
                srcb2[s, j, pl.ds(g * 16, 16)] = (
                    srcb[s, j, pl.ds(g * 16, 16)] + rowoff)
            pltpu.async_copy(xl_hbm.at[srcb2.at[s, j]], rows.at[s, j],
                             semG[s][j])

    def process(k, s):
        for j in range(_K):
            pltpu.make_async_copy(xl_hbm.at[srcb2.at[s, j]], rows.at[s, j],
                                  semG[s][j]).wait()
            for g in range(_CH // 16):
                attg = wv[s, j, pl.ds(g * 16, 16)]
                for t in range(16):
                    e = g * 16 + t
                    msg[s, j, e, pl.ds(0, 16)] = (
                        rows[s, j, e, pl.ds(0, 16)] * attg[t])
            pltpu.sync_copy(msg.at[s, j], acc_sh.at[dstb.at[s, j]], add=True)

    for sweep in range(2):
        rowoff = (2 * sweep + cid) * NP

        pltpu.sync_copy(zero_hbm, acc_sh.at[pl.ds(nbase, _NPT), :])
        plsc.subcore_barrier()

        issue_loads(0, 0)
        drain_loads(0, 0)
        issue_gathers(0, rowoff)
        issue_loads(1, 1)

        def body(t, _):
            for h in range(2):
                k = 2 * t + h

                @pl.when(k + 1 < n)
                def _():
                    drain_loads(k + 1, 1 - h)
                    issue_gathers(1 - h, rowoff)

                process(k, h)

                @pl.when(k + 2 < n)
                def _():
                    issue_loads(k + 2, h)
            return 0

        lax.fori_loop(0, n // 2, body, 0)
        plsc.subcore_barrier()
        pltpu.sync_copy(acc_sh.at[pl.ds(nbase, _NPT), :],
                        out_hbm.at[pl.ds(rowoff + nbase, _NPT), :])
        if sweep == 0:
            plsc.subcore_barrier()


_Z32 = None


def _gat_edges(src, dst, s, d, xl2):
    w, dpart = _edge_pass1(src, dst, s, d)
    dinv = 1.0 / (dpart[0] + dpart[1] + 1e-16)
    zero16 = jnp.zeros((_NPT, 16), F32)
    out = _edge_pass2(src, dst, w, zero16, xl2.reshape(4 * NP, 16))
    return out.reshape(4, NP, 16), dinv


def kernel(x, edge_index, batch, params):
    src = jnp.concatenate([edge_index[0],
                           jnp.zeros((EP - E,), jnp.int32)])
    dst = jnp.concatenate([edge_index[1],
                           jnp.full((EP - E,), N, jnp.int32)])
    batch_pad = jnp.concatenate([batch, jnp.full((NP - N,), G, jnp.int32)])
    xp = jnp.pad(x, ((0, NP - N), (0, 0)))

    p0 = params['layers'][0]
    xc, xl2, s, d, hh = _in_pre(
        xp, params['lin_in_W'].T, params['lin_in_b'],
        p0['gat_W'].T, p0['att_src'][0, 0], p0['att_dst'][0, 0],
        p0['h_W'].T, p0['h_b'])
    xloc = jnp.zeros((NP, D), F32)
    for i in range(L):
        p = params['layers'][i]
        conv, dinv = _gat_edges(src, dst, s, d, xl2)
        if i < L - 1:
            pn = params['layers'][i + 1]
            xc, xloc, xl2, s, d, hh = _post_pre(
                conv, dinv, xc, hh, xloc, p['lin_W'].T, p['lin_b'],
                p['gat_b'], p['ln_w'], p['ln_b'], params['betas'][i],
                pn['gat_W'].T, pn['att_src'][0, 0], pn['att_dst'][0, 0],
                pn['h_W'].T, pn['h_b'])
        else:
            out = _post_pool(
                conv, dinv, xc, hh, xloc, p['lin_W'].T, p['lin_b'],
                p['gat_b'], p['ln_w'], p['ln_b'], params['betas'][i],
                batch_pad, params['pred_W'].T, params['pred_b'])
    return out


# speedup vs baseline: 28.4047x; 1.0065x over previous
import functools

import jax
import jax.numpy as jnp
from jax import lax
from jax.experimental import pallas as pl
from jax.experimental.pallas import tpu as pltpu
from jax.experimental.pallas import tpu_sc as plsc

N = 50000
E = 800000
IN = 128
D = 64
G = 64
L = 3

NP = 51200
EP = 819200
BR = 1024
GRID = NP // BR

F32 = jnp.float32


def _pre_part(xc, gwt_ref, asrc_ref, adst_ref, hwt_ref, hb_ref,
              xl_ref, s_ref, d_ref, hh_ref):
    xl = jnp.dot(xc, gwt_ref[...], preferred_element_type=F32)
    for q in range(4):
        xl_ref[q] = xl[:, 16 * q:16 * q + 16]
    s_ref[...] = jnp.sum(xl * asrc_ref[...][None, :], axis=1)
    d_ref[...] = jnp.sum(xl * adst_ref[...][None, :], axis=1)
    hh_ref[...] = jax.nn.relu(
        jnp.dot(xc, hwt_ref[...], preferred_element_type=F32) + hb_ref[...])


def _post_part(conv_ref, dinv_ref, xc_ref, hh_ref, xloc_ref, lwt_ref, lb_ref,
               gb_ref, lnw_ref, lnb_ref, beta_ref):
    conv = (jnp.concatenate([conv_ref[q] for q in range(4)], axis=1)
            * dinv_ref[...][:, None] + gb_ref[...])
    t = jnp.dot(xc_ref[...], lwt_ref[...],
                preferred_element_type=F32) + lb_ref[...] + conv
    xc2 = jax.nn.relu(t)
    z = hh_ref[...] * xc2
    mu = jnp.mean(z, axis=-1, keepdims=True)
    var = jnp.mean((z - mu) ** 2, axis=-1, keepdims=True)
    ln = (z - mu) / jnp.sqrt(var + 1e-5) * lnw_ref[...] + lnb_ref[...]
    beta = jax.nn.sigmoid(beta_ref[...])[None, :]
    xcn = (1.0 - beta) * ln + beta * xc2
    return xcn, xloc_ref[...] + xcn


_VSPEC = pl.BlockSpec((D,), lambda i: (0,))
_MSPEC = pl.BlockSpec((BR, D), lambda i: (i, 0))
_WSPEC = pl.BlockSpec((D, D), lambda i: (0, 0))
_SSPEC = pl.BlockSpec((BR,), lambda i: (i,))
_XLSPEC = pl.BlockSpec((4, BR, 16), lambda i: (0, i, 0))


def _in_pre_body(x_ref, wt_ref, b_ref, gwt_ref, asrc_ref, adst_ref, hwt_ref,
                 hb_ref, xc_ref, xl_ref, s_ref, d_ref, hh_ref):
    xc = jnp.dot(x_ref[...], wt_ref[...],
                 preferred_element_type=F32) + b_ref[...]
    xc_ref[...] = xc
    _pre_part(xc, gwt_ref, asrc_ref, adst_ref, hwt_ref, hb_ref,
              xl_ref, s_ref, d_ref, hh_ref)


def _in_pre(x, wt, b, gwt, asrc, adst, hwt, hb):
    return pl.pallas_call(
        _in_pre_body,
        grid=(GRID,),
        in_specs=[
            pl.BlockSpec((BR, IN), lambda i: (i, 0)),
            pl.BlockSpec((IN, D), lambda i: (0, 0)),
            _VSPEC, _WSPEC, _VSPEC, _VSPEC, _WSPEC, _VSPEC,
        ],
        out_specs=[_MSPEC, _XLSPEC, _SSPEC, _SSPEC, _MSPEC],
        out_shape=[
            jax.ShapeDtypeStruct((NP, D), F32),
            jax.ShapeDtypeStruct((4, NP, 16), F32),
            jax.ShapeDtypeStruct((NP,), F32),
            jax.ShapeDtypeStruct((NP,), F32),
            jax.ShapeDtypeStruct((NP, D), F32),
        ],
    )(x, wt, b, gwt, asrc, adst, hwt, hb)


def _post_pre_body(conv_ref, dinv_ref, xc_ref, hh_ref, xloc_ref, lwt_ref,
                   lb_ref, gb_ref, lnw_ref, lnb_ref, beta_ref,
                   gwt_ref, asrc_ref, adst_ref, hwt_ref, hb_ref,
                   xcn_ref, xlocn_ref, xl_ref, s_ref, d_ref, hh2_ref):
    xcn, xlocn = _post_part(conv_ref, dinv_ref, xc_ref, hh_ref, xloc_ref,
                            lwt_ref, lb_ref, gb_ref, lnw_ref, lnb_ref,
                            beta_ref)
    xcn_ref[...] = xcn
    xlocn_ref[...] = xlocn
    _pre_part(xcn, gwt_ref, asrc_ref, adst_ref, hwt_ref, hb_ref,
              xl_ref, s_ref, d_ref, hh2_ref)


def _post_pre(conv, dinv, xc, hh, xloc, lwt, lb, gb, lnw, lnb, beta,
              gwt, asrc, adst, hwt, hb):
    return pl.pallas_call(
        _post_pre_body,
        grid=(GRID,),
        in_specs=[
            _XLSPEC, _SSPEC, _MSPEC, _MSPEC, _MSPEC,
            _WSPEC, _VSPEC, _VSPEC, _VSPEC, _VSPEC, _VSPEC,
            _WSPEC, _VSPEC, _VSPEC, _WSPEC, _VSPEC,
        ],
        out_specs=[_MSPEC, _MSPEC, _XLSPEC, _SSPEC, _SSPEC, _MSPEC],
        out_shape=[
            jax.ShapeDtypeStruct((NP, D), F32),
            jax.ShapeDtypeStruct((NP, D), F32),
            jax.ShapeDtypeStruct((4, NP, 16), F32),
            jax.ShapeDtypeStruct((NP,), F32),
            jax.ShapeDtypeStruct((NP,), F32),
            jax.ShapeDtypeStruct((NP, D), F32),
        ],
    )(conv, dinv, xc, hh, xloc, lwt, lb, gb, lnw, lnb, beta,
      gwt, asrc, adst, hwt, hb)


def _post_pool_body(conv_ref, dinv_ref, xc_ref, hh_ref, xloc_ref, lwt_ref,
                    lb_ref, gb_ref, lnw_ref, lnb_ref, beta_ref,
                    batch_ref, pwt_ref, pb_ref, o_ref, acc_s, acc_c):
    pid = pl.program_id(0)

    @pl.when(pid == 0)
    def _():
        acc_s[...] = jnp.zeros((G, D), F32)
        acc_c[...] = jnp.zeros((G, D), F32)

    _, xlocn = _post_part(conv_ref, dinv_ref, xc_ref, hh_ref, xloc_ref,
                          lwt_ref, lb_ref, gb_ref, lnw_ref, lnb_ref,
                          beta_ref)
    b = batch_ref[...]
    gids = lax.broadcasted_iota(jnp.int32, (G, BR), 0)
    mask = (b[None, :] == gids).astype(F32)
    acc_s[...] += jnp.dot(mask, xlocn, preferred_element_type=F32)
    cnt = jnp.sum(mask, axis=1)
    acc_c[...] += jnp.broadcast_to(cnt[:, None], (G, D))

    @pl.when(pid == GRID - 1)
    def _():
        pooled = acc_s[...] / jnp.maximum(acc_c[...], 1.0)
        o_ref[...] = jnp.dot(pooled, pwt_ref[...],
                             preferred_element_type=F32) + pb_ref[...]


def _post_pool(conv, dinv, xc, hh, xloc, lwt, lb, gb, lnw, lnb, beta,
               batch_pad, pwt, pb):
    return pl.pallas_call(
        _post_pool_body,
        grid=(GRID,),
        in_specs=[
            _XLSPEC, _SSPEC, _MSPEC, _MSPEC, _MSPEC,
            _WSPEC, _VSPEC, _VSPEC, _VSPEC, _VSPEC, _VSPEC,
            _SSPEC,
            pl.BlockSpec((D, G), lambda i: (0, 0)),
            pl.BlockSpec((G,), lambda i: (0,)),
        ],
        out_specs=pl.BlockSpec((G, G), lambda i: (0, 0)),
        out_shape=jax.ShapeDtypeStruct((G, G), F32),
        scratch_shapes=[pltpu.VMEM((G, D), F32), pltpu.VMEM((G, D), F32)],
    )(conv, dinv, xc, hh, xloc, lwt, lb, gb, lnw, lnb, beta,
      batch_pad, pwt, pb)


_NC = 2
_NS = 16
_CH = 128
_NPT = NP // _NS
_P1_PER = EP // (_NC * _NS)
_P1_CHUNKS = _P1_PER // _CH
_P2_PER = EP // _NS
_P2_CHUNKS = _P2_PER // _CH

_SCMESH = plsc.VectorSubcoreMesh(core_axis_name="c", subcore_axis_name="s",
                                 num_cores=_NC, num_subcores=_NS)
_K = 4


@functools.partial(
    pl.kernel,
    out_type=[jax.ShapeDtypeStruct((EP,), F32),
              jax.ShapeDtypeStruct((_NC, NP), F32)],
    mesh=_SCMESH,
    compiler_params=pltpu.CompilerParams(use_tc_tiling_on_sc=False),
    scratch_types=[
        pltpu.VMEM((2, _K * _CH), jnp.int32),
        pltpu.VMEM((2, _K * _CH), jnp.int32),
        pltpu.VMEM((2, _K * _CH), F32),
        pltpu.VMEM((2, _K * _CH), F32),
        pltpu.VMEM((2, _K * _CH), F32),
        pltpu.VMEM((_NPT,), F32),
        pltpu.VMEM_SHARED((NP,), F32),
        pltpu.SemaphoreType.DMA,
        pltpu.SemaphoreType.DMA,
        pltpu.SemaphoreType.DMA,
        pltpu.SemaphoreType.DMA,
        pltpu.SemaphoreType.DMA,
        pltpu.SemaphoreType.DMA,
        pltpu.SemaphoreType.DMA,
        pltpu.SemaphoreType.DMA,
        pltpu.SemaphoreType.DMA,
        pltpu.SemaphoreType.DMA,
        pltpu.SemaphoreType.DMA,
        pltpu.SemaphoreType.DMA,
    ])
def _edge_pass1(src_hbm, dst_hbm, s_hbm, d_hbm, w_hbm, dpart_hbm,
                srcb, dstb, sv, dv, wv, bounce, den_sh,
                semA0, semA1, sg00, sg01, sg02, sg03, sg10, sg11, sg12, sg13,
                semW0, semW1):
    cid = lax.axis_index("c")
    sid = lax.axis_index("s")
    wid = cid * _NS + sid
    nbase = sid * _NPT
    semA = [semA0, semA1]
    semG = [[sg00, sg01, sg02, sg03], [sg10, sg11, sg12, sg13]]
    semW = [semW0, semW1]

    zero = jnp.zeros((16,), F32)

    def zbody(i, _):
        bounce[pl.ds(i * 16, 16)] = zero
        return 0

    lax.fori_loop(0, _NPT // 16, zbody, 0)
    pltpu.sync_copy(bounce, den_sh.at[pl.ds(nbase, _NPT)])
    plsc.subcore_barrier()

    ebase0 = wid * _P1_PER
    n = _P1_CHUNKS // _K

    def issue_loads(k, s):
        gb = ebase0 + k * (_K * _CH)
        for j in range(_K):
            eb = gb + j * _CH
            pltpu.async_copy(src_hbm.at[pl.ds(eb, _CH)], srcb.at[s, pl.ds(j * _CH, _CH)],
                             semA[s])
            pltpu.async_copy(dst_hbm.at[pl.ds(eb, _CH)], dstb.at[s, pl.ds(j * _CH, _CH)],
                             semA[s])

    def drain_loads(k, s):
        gb = ebase0 + k * (_K * _CH)
        for j in range(_K):
            eb = gb + j * _CH
            pltpu.make_async_copy(src_hbm.at[pl.ds(eb, _CH)], srcb.at[s, pl.ds(j * _CH, _CH)],
                                  semA[s]).wait()
            pltpu.make_async_copy(dst_hbm.at[pl.ds(eb, _CH)], dstb.at[s, pl.ds(j * _CH, _CH)],
                                  semA[s]).wait()

    def issue_gathers(s):
        for j in range(_K):
            pltpu.async_copy(s_hbm.at[srcb.at[s, pl.ds(j * _CH, _CH)]], sv.at[s, pl.ds(j * _CH, _CH)], semG[s][j])
            pltpu.async_copy(d_hbm.at[dstb.at[s, pl.ds(j * _CH, _CH)]], dv.at[s, pl.ds(j * _CH, _CH)], semG[s][j])

    def drain_writes(k, s):
        gb = ebase0 + k * (_K * _CH)
        for j in range(_K):
            pltpu.make_async_copy(wv.at[s, pl.ds(j * _CH, _CH)],
                                  w_hbm.at[pl.ds(gb + j * _CH, _CH)],
                                  semW[s]).wait()

    def process(k, s):
        gb = ebase0 + k * (_K * _CH)

        @pl.when(k >= 2)
        def _():
            drain_writes(k - 2, s)

        for j in range(_K):
            pltpu.make_async_copy(s_hbm.at[srcb.at[s, pl.ds(j * _CH, _CH)]], sv.at[s, pl.ds(j * _CH, _CH)],
                                  semG[s][j]).wait()
            pltpu.make_async_copy(d_hbm.at[dstb.at[s, pl.ds(j * _CH, _CH)]], dv.at[s, pl.ds(j * _CH, _CH)],
                                  semG[s][j]).wait()
            for g in range(_CH // 16):
                a = sv[s, pl.ds(j * _CH + g * 16, 16)] + dv[s, pl.ds(j * _CH + g * 16, 16)]
                a = jnp.maximum(a, 0.0) + 0.2 * jnp.minimum(a, 0.0)
                wv[s, pl.ds(j * _CH + g * 16, 16)] = jnp.exp(a)
            pltpu.async_copy(wv.at[s, pl.ds(j * _CH, _CH)], w_hbm.at[pl.ds(gb + j * _CH, _CH)],
                             semW[s])
        pltpu.sync_copy(wv.at[s], den_sh.at[dstb.at[s]], add=True)

    issue_loads(0, 0)
    drain_loads(0, 0)
    issue_gathers(0)
    issue_loads(1, 1)

    def body(t, _):
        for h in range(2):
            k = 2 * t + h

            @pl.when(k + 1 < n)
            def _():
                drain_loads(k + 1, 1 - h)
                issue_gathers(1 - h)

            process(k, h)

            @pl.when(k + 2 < n)
            def _():
                issue_loads(k + 2, h)
        return 0

    lax.fori_loop(0, n // 2, body, 0)
    drain_writes(n - 2, 0)
    drain_writes(n - 1, 1)
    plsc.subcore_barrier()
    pltpu.sync_copy(den_sh.at[pl.ds(nbase, _NPT)], bounce)
    pltpu.sync_copy(bounce, dpart_hbm.at[cid].at[pl.ds(nbase, _NPT)])


@functools.partial(
    pl.kernel,
    out_type=jax.ShapeDtypeStruct((4 * NP, 16), F32),
    mesh=_SCMESH,
    compiler_params=pltpu.CompilerParams(use_tc_tiling_on_sc=False),
    scratch_types=[
        pltpu.VMEM((2, _K * _CH), jnp.int32),
        pltpu.VMEM((2, _K * _CH), jnp.int32),
        pltpu.VMEM((2, _K * _CH), jnp.int32),
        pltpu.VMEM((2, _K * _CH), F32),
        pltpu.VMEM((2, _K * _CH, 16), F32),
        pltpu.VMEM((2, _K * _CH, 16), F32),
        pltpu.VMEM_SHARED((NP, 16), F32),
        pltpu.SemaphoreType.DMA,
        pltpu.SemaphoreType.DMA,
        pltpu.SemaphoreType.DMA,
        pltpu.SemaphoreType.DMA,
        pltpu.SemaphoreType.DMA,
        pltpu.SemaphoreType.DMA,
        pltpu.SemaphoreType.DMA,
        pltpu.SemaphoreType.DMA,
        pltpu.SemaphoreType.DMA,
        pltpu.SemaphoreType.DMA,
    ])
def _edge_pass2(src_hbm, dst_hbm, w_hbm, zero_hbm, xl_hbm, out_hbm,
                srcb, dstb, srcb2, wv, rows, msg, acc_sh,
                semA0, semA1, sg00, sg01, sg02, sg03, sg10, sg11, sg12, sg13):
    cid = lax.axis_index("c")
    sid = lax.axis_index("s")
    nbase = sid * _NPT
    semA = [semA0, semA1]
    semG = [[sg00, sg01, sg02, sg03], [sg10, sg11, sg12, sg13]]
    ebase0 = sid * _P2_PER
    n = _P2_CHUNKS // _K

    def issue_loads(k, s):
        gb = ebase0 + k * (_K * _CH)
        for j in range(_K):
            eb = gb + j * _CH
            pltpu.async_copy(src_hbm.at[pl.ds(eb, _CH)], srcb.at[s, pl.ds(j * _CH, _CH)],
                             semA[s])
            pltpu.async_copy(dst_hbm.at[pl.ds(eb, _CH)], dstb.at[s, pl.ds(j * _CH, _CH)],
                             semA[s])
            pltpu.async_copy(w_hbm.at[pl.ds(eb, _CH)], wv.at[s, pl.ds(j * _CH, _CH)], semA[s])

    def drain_loads(k, s):
        gb = ebase0 + k * (_K * _CH)
        for j in range(_K):
            eb = gb + j * _CH
            pltpu.make_async_copy(src_hbm.at[pl.ds(eb, _CH)], srcb.at[s, pl.ds(j * _CH, _CH)],
                                  semA[s]).wait()
            pltpu.make_async_copy(dst_hbm.at[pl.ds(eb, _CH)], dstb.at[s, pl.ds(j * _CH, _CH)],
                                  semA[s]).wait()
            pltpu.make_async_copy(w_hbm.at[pl.ds(eb, _CH)], wv.at[s, pl.ds(j * _CH, _CH)],
                                  semA[s]).wait()

    def issue_gathers(s, rowoff):
        for j in range(_K):
            for g in range(_CH // 16):
                srcb2[s, pl.ds(j * _CH + g * 16, 16)] = (
                    srcb[s, pl.ds(j * _CH + g * 16, 16)] + rowoff)
            pltpu.async_copy(xl_hbm.at[srcb2.at[s, pl.ds(j * _CH, _CH)]], rows.at[s, pl.ds(j * _CH, _CH), :],
                             semG[s][j])

    def process(k, s):
        for j in range(_K):
            pltpu.make_async_copy(xl_hbm.at[srcb2.at[s, pl.ds(j * _CH, _CH)]], rows.at[s, pl.ds(j * _CH, _CH), :],
                                  semG[s][j]).wait()
            for g in range(_CH // 16):
                attg = wv[s, pl.ds(j * _CH + g * 16, 16)]
                for t in range(16):
                    e = g * 16 + t
                    msg[s, j * _CH + e, pl.ds(0, 16)] = (
                        rows[s, j * _CH + e, pl.ds(0, 16)] * attg[t])
        pltpu.sync_copy(msg.at[s], acc_sh.at[dstb.at[s]], add=True)

    for sweep in range(2):
        rowoff = (2 * sweep + cid) * NP

        pltpu.sync_copy(zero_hbm, acc_sh.at[pl.ds(nbase, _NPT), :])
        plsc.subcore_barrier()

        issue_loads(0, 0)
        drain_loads(0, 0)
        issue_gathers(0, rowoff)
        issue_loads(1, 1)

        def body(t, _):
            for h in range(2):
                k = 2 * t + h

                @pl.when(k + 1 < n)
                def _():
                    drain_loads(k + 1, 1 - h)
                    issue_gathers(1 - h, rowoff)

                process(k, h)

                @pl.when(k + 2 < n)
                def _():
                    issue_loads(k + 2, h)
            return 0

        lax.fori_loop(0, n // 2, body, 0)
        plsc.subcore_barrier()
        pltpu.sync_copy(acc_sh.at[pl.ds(nbase, _NPT), :],
                        out_hbm.at[pl.ds(rowoff + nbase, _NPT), :])
        if sweep == 0:
            plsc.subcore_barrier()


_Z32 = None


def _gat_edges(src, dst, s, d, xl2):
    w, dpart = _edge_pass1(src, dst, s, d)
    dinv = 1.0 / (dpart[0] + dpart[1] + 1e-16)
    zero16 = jnp.zeros((_NPT, 16), F32)
    out = _edge_pass2(src, dst, w, zero16, xl2.reshape(4 * NP, 16))
    return out.reshape(4, NP, 16), dinv


def kernel(x, edge_index, batch, params):
    src = jnp.concatenate([edge_index[0],
                           jnp.zeros((EP - E,), jnp.int32)])
    dst = jnp.concatenate([edge_index[1],
                           jnp.full((EP - E,), N, jnp.int32)])
    batch_pad = jnp.concatenate([batch, jnp.full((NP - N,), G, jnp.int32)])
    xp = jnp.pad(x, ((0, NP - N), (0, 0)))

    p0 = params['layers'][0]
    xc, xl2, s, d, hh = _in_pre(
        xp, params['lin_in_W'].T, params['lin_in_b'],
        p0['gat_W'].T, p0['att_src'][0, 0], p0['att_dst'][0, 0],
        p0['h_W'].T, p0['h_b'])
    xloc = jnp.zeros((NP, D), F32)
    for i in range(L):
        p = params['layers'][i]
        conv, dinv = _gat_edges(src, dst, s, d, xl2)
        if i < L - 1:
            pn = params['layers'][i + 1]
            xc, xloc, xl2, s, d, hh = _post_pre(
                conv, dinv, xc, hh, xloc, p['lin_W'].T, p['lin_b'],
                p['gat_b'], p['ln_w'], p['ln_b'], params['betas'][i],
                pn['gat_W'].T, pn['att_src'][0, 0], pn['att_dst'][0, 0],
                pn['h_W'].T, pn['h_b'])
        else:
            out = _post_pool(
                conv, dinv, xc, hh, xloc, p['lin_W'].T, p['lin_b'],
                p['gat_b'], p['ln_w'], p['ln_b'], params['betas'][i],
                batch_pad, params['pred_W'].T, params['pred_b'])
    return out


# speedup vs baseline: 29.3182x; 1.0322x over previous
import functools

import jax
import jax.numpy as jnp
from jax import lax
from jax.experimental import pallas as pl
from jax.experimental.pallas import tpu as pltpu
from jax.experimental.pallas import tpu_sc as plsc

N = 50000
E = 800000
IN = 128
D = 64
G = 64
L = 3

NP = 51200
EP = 819200
BR = 1024
GRID = NP // BR

F32 = jnp.float32


def _pre_part(xc, gwt_ref, asrc_ref, adst_ref, hwt_ref, hb_ref,
              xl_ref, s_ref, d_ref, hh_ref):
    xl = jnp.dot(xc, gwt_ref[...], preferred_element_type=F32)
    for q in range(4):
        xl_ref[q] = xl[:, 16 * q:16 * q + 16]
    s_ref[...] = jnp.sum(xl * asrc_ref[...][None, :], axis=1)
    d_ref[...] = jnp.sum(xl * adst_ref[...][None, :], axis=1)
    hh_ref[...] = jax.nn.relu(
        jnp.dot(xc, hwt_ref[...], preferred_element_type=F32) + hb_ref[...])


def _post_part(conv_ref, dinv_ref, xc_ref, hh_ref, xloc_ref, lwt_ref, lb_ref,
               gb_ref, lnw_ref, lnb_ref, beta_ref):
    dinv = 1.0 / (dinv_ref[0] + dinv_ref[1] + 1e-16)
    conv = (jnp.concatenate([conv_ref[q] for q in range(4)], axis=1)
            * dinv[:, None] + gb_ref[...])
    t = jnp.dot(xc_ref[...], lwt_ref[...],
                preferred_element_type=F32) + lb_ref[...] + conv
    xc2 = jax.nn.relu(t)
    z = hh_ref[...] * xc2
    mu = jnp.mean(z, axis=-1, keepdims=True)
    var = jnp.mean((z - mu) ** 2, axis=-1, keepdims=True)
    ln = (z - mu) / jnp.sqrt(var + 1e-5) * lnw_ref[...] + lnb_ref[...]
    beta = jax.nn.sigmoid(beta_ref[...])[None, :]
    xcn = (1.0 - beta) * ln + beta * xc2
    return xcn, xloc_ref[...] + xcn


_VSPEC = pl.BlockSpec((D,), lambda i: (0,))
_MSPEC = pl.BlockSpec((BR, D), lambda i: (i, 0))
_WSPEC = pl.BlockSpec((D, D), lambda i: (0, 0))
_SSPEC = pl.BlockSpec((BR,), lambda i: (i,))
_DPSPEC = pl.BlockSpec((2, BR), lambda i: (0, i))
_XLSPEC = pl.BlockSpec((4, BR, 16), lambda i: (0, i, 0))


def _in_pre_body(x_ref, wt_ref, b_ref, gwt_ref, asrc_ref, adst_ref, hwt_ref,
                 hb_ref, xc_ref, xl_ref, s_ref, d_ref, hh_ref):
    xc = jnp.dot(x_ref[...], wt_ref[...],
                 preferred_element_type=F32) + b_ref[...]
    xc_ref[...] = xc
    _pre_part(xc, gwt_ref, asrc_ref, adst_ref, hwt_ref, hb_ref,
              xl_ref, s_ref, d_ref, hh_ref)


def _in_pre(x, wt, b, gwt, asrc, adst, hwt, hb):
    return pl.pallas_call(
        _in_pre_body,
        grid=(GRID,),
        in_specs=[
            pl.BlockSpec((BR, IN), lambda i: (i, 0)),
            pl.BlockSpec((IN, D), lambda i: (0, 0)),
            _VSPEC, _WSPEC, _VSPEC, _VSPEC, _WSPEC, _VSPEC,
        ],
        out_specs=[_MSPEC, _XLSPEC, _SSPEC, _SSPEC, _MSPEC],
        out_shape=[
            jax.ShapeDtypeStruct((NP, D), F32),
            jax.ShapeDtypeStruct((4, NP, 16), F32),
            jax.ShapeDtypeStruct((NP,), F32),
            jax.ShapeDtypeStruct((NP,), F32),
            jax.ShapeDtypeStruct((NP, D), F32),
        ],
    )(x, wt, b, gwt, asrc, adst, hwt, hb)


def _post_pre_body(conv_ref, dinv_ref, xc_ref, hh_ref, xloc_ref, lwt_ref,
                   lb_ref, gb_ref, lnw_ref, lnb_ref, beta_ref,
                   gwt_ref, asrc_ref, adst_ref, hwt_ref, hb_ref,
                   xcn_ref, xlocn_ref, xl_ref, s_ref, d_ref, hh2_ref):
    xcn, xlocn = _post_part(conv_ref, dinv_ref, xc_ref, hh_ref, xloc_ref,
                            lwt_ref, lb_ref, gb_ref, lnw_ref, lnb_ref,
                            beta_ref)
    xcn_ref[...] = xcn
    xlocn_ref[...] = xlocn
    _pre_part(xcn, gwt_ref, asrc_ref, adst_ref, hwt_ref, hb_ref,
              xl_ref, s_ref, d_ref, hh2_ref)


def _post_pre(conv, dinv, xc, hh, xloc, lwt, lb, gb, lnw, lnb, beta,
              gwt, asrc, adst, hwt, hb):
    return pl.pallas_call(
        _post_pre_body,
        grid=(GRID,),
        in_specs=[
            _XLSPEC, _DPSPEC, _MSPEC, _MSPEC, _MSPEC,
            _WSPEC, _VSPEC, _VSPEC, _VSPEC, _VSPEC, _VSPEC,
            _WSPEC, _VSPEC, _VSPEC, _WSPEC, _VSPEC,
        ],
        out_specs=[_MSPEC, _MSPEC, _XLSPEC, _SSPEC, _SSPEC, _MSPEC],
        out_shape=[
            jax.ShapeDtypeStruct((NP, D), F32),
            jax.ShapeDtypeStruct((NP, D), F32),
            jax.ShapeDtypeStruct((4, NP, 16), F32),
            jax.ShapeDtypeStruct((NP,), F32),
            jax.ShapeDtypeStruct((NP,), F32),
            jax.ShapeDtypeStruct((NP, D), F32),
        ],
    )(conv, dinv, xc, hh, xloc, lwt, lb, gb, lnw, lnb, beta,
      gwt, asrc, adst, hwt, hb)


def _post_pool_body(conv_ref, dinv_ref, xc_ref, hh_ref, xloc_ref, lwt_ref,
                    lb_ref, gb_ref, lnw_ref, lnb_ref, beta_ref,
                    batch_ref, pwt_ref, pb_ref, o_ref, acc_s, acc_c):
    pid = pl.program_id(0)

    @pl.when(pid == 0)
    def _():
        acc_s[...] = jnp.zeros((G, D), F32)
        acc_c[...] = jnp.zeros((G, D), F32)

    _, xlocn = _post_part(conv_ref, dinv_ref, xc_ref, hh_ref, xloc_ref,
                          lwt_ref, lb_ref, gb_ref, lnw_ref, lnb_ref,
                          beta_ref)
    b = batch_ref[...]
    gids = lax.broadcasted_iota(jnp.int32, (G, BR), 0)
    mask = (b[None, :] == gids).astype(F32)
    acc_s[...] += jnp.dot(mask, xlocn, preferred_element_type=F32)
    cnt = jnp.sum(mask, axis=1)
    acc_c[...] += jnp.broadcast_to(cnt[:, None], (G, D))

    @pl.when(pid == GRID - 1)
    def _():
        pooled = acc_s[...] / jnp.maximum(acc_c[...], 1.0)
        o_ref[...] = jnp.dot(pooled, pwt_ref[...],
                             preferred_element_type=F32) + pb_ref[...]


def _post_pool(conv, dinv, xc, hh, xloc, lwt, lb, gb, lnw, lnb, beta,
               batch_pad, pwt, pb):
    return pl.pallas_call(
        _post_pool_body,
        grid=(GRID,),
        in_specs=[
            _XLSPEC, _DPSPEC, _MSPEC, _MSPEC, _MSPEC,
            _WSPEC, _VSPEC, _VSPEC, _VSPEC, _VSPEC, _VSPEC,
            _SSPEC,
            pl.BlockSpec((D, G), lambda i: (0, 0)),
            pl.BlockSpec((G,), lambda i: (0,)),
        ],
        out_specs=pl.BlockSpec((G, G), lambda i: (0, 0)),
        out_shape=jax.ShapeDtypeStruct((G, G), F32),
        scratch_shapes=[pltpu.VMEM((G, D), F32), pltpu.VMEM((G, D), F32)],
    )(conv, dinv, xc, hh, xloc, lwt, lb, gb, lnw, lnb, beta,
      batch_pad, pwt, pb)


_NC = 2
_NS = 16
_CH = 128
_NPT = NP // _NS
_P1_PER = EP // (_NC * _NS)
_P1_CHUNKS = _P1_PER // _CH
_P2_PER = EP // _NS
_P2_CHUNKS = _P2_PER // _CH

_SCMESH = plsc.VectorSubcoreMesh(core_axis_name="c", subcore_axis_name="s",
                                 num_cores=_NC, num_subcores=_NS)
_K = 4


@functools.partial(
    pl.kernel,
    out_type=[jax.ShapeDtypeStruct((EP,), F32),
              jax.ShapeDtypeStruct((_NC, NP), F32)],
    mesh=_SCMESH,
    compiler_params=pltpu.CompilerParams(use_tc_tiling_on_sc=False),
    scratch_types=[
        pltpu.VMEM((2, _K * _CH), jnp.int32),
        pltpu.VMEM((2, _K * _CH), jnp.int32),
        pltpu.VMEM((2, _K * _CH), F32),
        pltpu.VMEM((2, _K * _CH), F32),
        pltpu.VMEM((2, _K * _CH), F32),
        pltpu.VMEM((_NPT,), F32),
        pltpu.VMEM_SHARED((NP,), F32),
        pltpu.SemaphoreType.DMA,
        pltpu.SemaphoreType.DMA,
        pltpu.SemaphoreType.DMA,
        pltpu.SemaphoreType.DMA,
        pltpu.SemaphoreType.DMA,
        pltpu.SemaphoreType.DMA,
        pltpu.SemaphoreType.DMA,
        pltpu.SemaphoreType.DMA,
        pltpu.SemaphoreType.DMA,
        pltpu.SemaphoreType.DMA,
        pltpu.SemaphoreType.DMA,
        pltpu.SemaphoreType.DMA,
    ])
def _edge_pass1(src_hbm, dst_hbm, s_hbm, d_hbm, w_hbm, dpart_hbm,
                srcb, dstb, sv, dv, wv, bounce, den_sh,
                semA0, semA1, sg00, sg01, sg02, sg03, sg10, sg11, sg12, sg13,
                semW0, semW1):
    cid = lax.axis_index("c")
    sid = lax.axis_index("s")
    wid = cid * _NS + sid
    nbase = sid * _NPT
    semA = [semA0, semA1]
    semG = [[sg00, sg01, sg02, sg03], [sg10, sg11, sg12, sg13]]
    semW = [semW0, semW1]

    zero = jnp.zeros((16,), F32)

    def zbody(i, _):
        bounce[pl.ds(i * 16, 16)] = zero
        return 0

    lax.fori_loop(0, _NPT // 16, zbody, 0)
    pltpu.sync_copy(bounce, den_sh.at[pl.ds(nbase, _NPT)])
    plsc.subcore_barrier()

    ebase0 = wid * _P1_PER
    n = _P1_CHUNKS // _K

    def issue_loads(k, s):
        gb = ebase0 + k * (_K * _CH)
        for j in range(_K):
            eb = gb + j * _CH
            pltpu.async_copy(src_hbm.at[pl.ds(eb, _CH)], srcb.at[s, pl.ds(j * _CH, _CH)],
                             semA[s])
            pltpu.async_copy(dst_hbm.at[pl.ds(eb, _CH)], dstb.at[s, pl.ds(j * _CH, _CH)],
                             semA[s])

    def drain_loads(k, s):
        gb = ebase0 + k * (_K * _CH)
        for j in range(_K):
            eb = gb + j * _CH
            pltpu.make_async_copy(src_hbm.at[pl.ds(eb, _CH)], srcb.at[s, pl.ds(j * _CH, _CH)],
                                  semA[s]).wait()
            pltpu.make_async_copy(dst_hbm.at[pl.ds(eb, _CH)], dstb.at[s, pl.ds(j * _CH, _CH)],
                                  semA[s]).wait()

    def issue_gathers(s):
        for j in range(_K):
            pltpu.async_copy(s_hbm.at[srcb.at[s, pl.ds(j * _CH, _CH)]], sv.at[s, pl.ds(j * _CH, _CH)], semG[s][j])
            pltpu.async_copy(d_hbm.at[dstb.at[s, pl.ds(j * _CH, _CH)]], dv.at[s, pl.ds(j * _CH, _CH)], semG[s][j])

    def drain_writes(k, s):
        gb = ebase0 + k * (_K * _CH)
        for j in range(_K):
            pltpu.make_async_copy(wv.at[s, pl.ds(j * _CH, _CH)],
                                  w_hbm.at[pl.ds(gb + j * _CH, _CH)],
                                  semW[s]).wait()

    def process(k, s):
        gb = ebase0 + k * (_K * _CH)

        @pl.when(k >= 2)
        def _():
            drain_writes(k - 2, s)

        for j in range(_K):
            pltpu.make_async_copy(s_hbm.at[srcb.at[s, pl.ds(j * _CH, _CH)]], sv.at[s, pl.ds(j * _CH, _CH)],
                                  semG[s][j]).wait()
            pltpu.make_async_copy(d_hbm.at[dstb.at[s, pl.ds(j * _CH, _CH)]], dv.at[s, pl.ds(j * _CH, _CH)],
                                  semG[s][j]).wait()
            for g in range(_CH // 16):
                a = sv[s, pl.ds(j * _CH + g * 16, 16)] + dv[s, pl.ds(j * _CH + g * 16, 16)]
                a = jnp.maximum(a, 0.0) + 0.2 * jnp.minimum(a, 0.0)
                wv[s, pl.ds(j * _CH + g * 16, 16)] = jnp.exp(a)
            pltpu.async_copy(wv.at[s, pl.ds(j * _CH, _CH)], w_hbm.at[pl.ds(gb + j * _CH, _CH)],
                             semW[s])
        pltpu.sync_copy(wv.at[s], den_sh.at[dstb.at[s]], add=True)

    issue_loads(0, 0)
    drain_loads(0, 0)
    issue_gathers(0)
    issue_loads(1, 1)

    def body(t, _):
        for h in range(2):
            k = 2 * t + h

            @pl.when(k + 1 < n)
            def _():
                drain_loads(k + 1, 1 - h)
                issue_gathers(1 - h)

            process(k, h)

            @pl.when(k + 2 < n)
            def _():
                issue_loads(k + 2, h)
        return 0

    lax.fori_loop(0, n // 2, body, 0)
    drain_writes(n - 2, 0)
    drain_writes(n - 1, 1)
    plsc.subcore_barrier()
    pltpu.sync_copy(den_sh.at[pl.ds(nbase, _NPT)], bounce)
    pltpu.sync_copy(bounce, dpart_hbm.at[cid].at[pl.ds(nbase, _NPT)])


@functools.partial(
    pl.kernel,
    out_type=jax.ShapeDtypeStruct((4 * NP, 16), F32),
    mesh=_SCMESH,
    compiler_params=pltpu.CompilerParams(use_tc_tiling_on_sc=False),
    scratch_types=[
        pltpu.VMEM((2, _K * _CH), jnp.int32),
        pltpu.VMEM((2, _K * _CH), jnp.int32),
        pltpu.VMEM((2, _K * _CH), jnp.int32),
        pltpu.VMEM((2, _K * _CH), F32),
        pltpu.VMEM((2, _K * _CH, 16), F32),
        pltpu.VMEM((2, _K * _CH, 16), F32),
        pltpu.VMEM_SHARED((NP, 16), F32),
        pltpu.SemaphoreType.DMA,
        pltpu.SemaphoreType.DMA,
        pltpu.SemaphoreType.DMA,
        pltpu.SemaphoreType.DMA,
        pltpu.SemaphoreType.DMA,
        pltpu.SemaphoreType.DMA,
        pltpu.SemaphoreType.DMA,
        pltpu.SemaphoreType.DMA,
        pltpu.SemaphoreType.DMA,
        pltpu.SemaphoreType.DMA,
    ])
def _edge_pass2(src_hbm, dst_hbm, w_hbm, zero_hbm, xl_hbm, out_hbm,
                srcb, dstb, srcb2, wv, rows, msg, acc_sh,
                semA0, semA1, sg00, sg01, sg02, sg03, sg10, sg11, sg12, sg13):
    cid = lax.axis_index("c")
    sid = lax.axis_index("s")
    nbase = sid * _NPT
    semA = [semA0, semA1]
    semG = [[sg00, sg01, sg02, sg03], [sg10, sg11, sg12, sg13]]
    ebase0 = sid * _P2_PER
    n = _P2_CHUNKS // _K

    def issue_loads(k, s):
        gb = ebase0 + k * (_K * _CH)
        for j in range(_K):
            eb = gb + j * _CH
            pltpu.async_copy(src_hbm.at[pl.ds(eb, _CH)], srcb.at[s, pl.ds(j * _CH, _CH)],
                             semA[s])
            pltpu.async_copy(dst_hbm.at[pl.ds(eb, _CH)], dstb.at[s, pl.ds(j * _CH, _CH)],
                             semA[s])
            pltpu.async_copy(w_hbm.at[pl.ds(eb, _CH)], wv.at[s, pl.ds(j * _CH, _CH)], semA[s])

    def drain_loads(k, s):
        gb = ebase0 + k * (_K * _CH)
        for j in range(_K):
            eb = gb + j * _CH
            pltpu.make_async_copy(src_hbm.at[pl.ds(eb, _CH)], srcb.at[s, pl.ds(j * _CH, _CH)],
                                  semA[s]).wait()
            pltpu.make_async_copy(dst_hbm.at[pl.ds(eb, _CH)], dstb.at[s, pl.ds(j * _CH, _CH)],
                                  semA[s]).wait()
            pltpu.make_async_copy(w_hbm.at[pl.ds(eb, _CH)], wv.at[s, pl.ds(j * _CH, _CH)],
                                  semA[s]).wait()

    def issue_gathers(s, rowoff):
        for j in range(_K):
            for g in range(_CH // 16):
                srcb2[s, pl.ds(j * _CH + g * 16, 16)] = (
                    srcb[s, pl.ds(j * _CH + g * 16, 16)] + rowoff)
            pltpu.async_copy(xl_hbm.at[srcb2.at[s, pl.ds(j * _CH, _CH)]], rows.at[s, pl.ds(j * _CH, _CH), :],
                             semG[s][j])

    def process(k, s):
        for j in range(_K):
            pltpu.make_async_copy(xl_hbm.at[srcb2.at[s, pl.ds(j * _CH, _CH)]], rows.at[s, pl.ds(j * _CH, _CH), :],
                                  semG[s][j]).wait()
            for g in range(_CH // 16):
                attg = wv[s, pl.ds(j * _CH + g * 16, 16)]
                for t in range(16):
                    e = g * 16 + t
                    msg[s, j * _CH + e, pl.ds(0, 16)] = (
                        rows[s, j * _CH + e, pl.ds(0, 16)] * attg[t])
        pltpu.sync_copy(msg.at[s], acc_sh.at[dstb.at[s]], add=True)

    for sweep in range(2):
        rowoff = (2 * sweep + cid) * NP

        pltpu.sync_copy(zero_hbm, acc_sh.at[pl.ds(nbase, _NPT), :])
        plsc.subcore_barrier()

        issue_loads(0, 0)
        drain_loads(0, 0)
        issue_gathers(0, rowoff)
        issue_loads(1, 1)

        def body(t, _):
            for h in range(2):
                k = 2 * t + h

                @pl.when(k + 1 < n)
                def _():
                    drain_loads(k + 1, 1 - h)
                    issue_gathers(1 - h, rowoff)

                process(k, h)

                @pl.when(k + 2 < n)
                def _():
                    issue_loads(k + 2, h)
            return 0

        lax.fori_loop(0, n // 2, body, 0)
        plsc.subcore_barrier()
        pltpu.sync_copy(acc_sh.at[pl.ds(nbase, _NPT), :],
                        out_hbm.at[pl.ds(rowoff + nbase, _NPT), :])
        if sweep == 0:
            plsc.subcore_barrier()


_Z32 = None


def _gat_edges(src, dst, s, d, xl2):
    w, dpart = _edge_pass1(src, dst, s, d)
    zero16 = jnp.zeros((_NPT, 16), F32)
    out = _edge_pass2(src, dst, w, zero16, xl2.reshape(4 * NP, 16))
    return out.reshape(4, NP, 16), dpart


def kernel(x, edge_index, batch, params):
    src = jnp.concatenate([edge_index[0],
                           jnp.zeros((EP - E,), jnp.int32)])
    dst = jnp.concatenate([edge_index[1],
                           jnp.full((EP - E,), N, jnp.int32)])
    batch_pad = jnp.concatenate([batch, jnp.full((NP - N,), G, jnp.int32)])
    xp = jnp.pad(x, ((0, NP - N), (0, 0)))

    p0 = params['layers'][0]
    xc, xl2, s, d, hh = _in_pre(
        xp, params['lin_in_W'].T, params['lin_in_b'],
        p0['gat_W'].T, p0['att_src'][0, 0], p0['att_dst'][0, 0],
        p0['h_W'].T, p0['h_b'])
    xloc = jnp.zeros((NP, D), F32)
    for i in range(L):
        p = params['layers'][i]
        conv, dinv = _gat_edges(src, dst, s, d, xl2)
        if i < L - 1:
            pn = params['layers'][i + 1]
            xc, xloc, xl2, s, d, hh = _post_pre(
                conv, dinv, xc, hh, xloc, p['lin_W'].T, p['lin_b'],
                p['gat_b'], p['ln_w'], p['ln_b'], params['betas'][i],
                pn['gat_W'].T, pn['att_src'][0, 0], pn['att_dst'][0, 0],
                pn['h_W'].T, pn['h_b'])
        else:
            out = _post_pool(
                conv, dinv, xc, hh, xloc, p['lin_W'].T, p['lin_b'],
                p['gat_b'], p['ln_w'], p['ln_b'], params['betas'][i],
                batch_pad, params['pred_W'].T, params['pred_b'])
    return out


# speedup vs baseline: 29.4571x; 1.0047x over previous
import functools

import jax
import jax.numpy as jnp
from jax import lax
from jax.experimental import pallas as pl
from jax.experimental.pallas import tpu as pltpu
from jax.experimental.pallas import tpu_sc as plsc

N = 50000
E = 800000
IN = 128
D = 64
G = 64
L = 3

NP = 51200
EP = 819200
BR = 1024
GRID = NP // BR

F32 = jnp.float32


def _pre_part(xc, gwt_ref, asrc_ref, adst_ref, hwt_ref, hb_ref,
              xl_ref, s_ref, d_ref, hh_ref):
    xl = jnp.dot(xc, gwt_ref[...], preferred_element_type=F32)
    for q in range(4):
        xl_ref[q] = xl[:, 16 * q:16 * q + 16]
    s_ref[...] = jnp.sum(xl * asrc_ref[...][None, :], axis=1)
    d_ref[...] = jnp.sum(xl * adst_ref[...][None, :], axis=1)
    hh_ref[...] = jax.nn.relu(
        jnp.dot(xc, hwt_ref[...], preferred_element_type=F32) + hb_ref[...])


def _post_part(conv_ref, dinv_ref, xc_ref, hh_ref, xloc_ref, lwt_ref, lb_ref,
               gb_ref, lnw_ref, lnb_ref, beta_ref):
    dinv = 1.0 / (dinv_ref[0] + dinv_ref[1] + 1e-16)
    conv = (jnp.concatenate([conv_ref[q] for q in range(4)], axis=1)
            * dinv[:, None] + gb_ref[...])
    t = jnp.dot(xc_ref[...], lwt_ref[...],
                preferred_element_type=F32) + lb_ref[...] + conv
    xc2 = jax.nn.relu(t)
    z = hh_ref[...] * xc2
    mu = jnp.mean(z, axis=-1, keepdims=True)
    var = jnp.mean((z - mu) ** 2, axis=-1, keepdims=True)
    ln = (z - mu) / jnp.sqrt(var + 1e-5) * lnw_ref[...] + lnb_ref[...]
    beta = jax.nn.sigmoid(beta_ref[...])[None, :]
    xcn = (1.0 - beta) * ln + beta * xc2
    return xcn, xloc_ref[...] + xcn


_VSPEC = pl.BlockSpec((D,), lambda i: (0,))
_MSPEC = pl.BlockSpec((BR, D), lambda i: (i, 0))
_WSPEC = pl.BlockSpec((D, D), lambda i: (0, 0))
_SSPEC = pl.BlockSpec((BR,), lambda i: (i,))
_DPSPEC = pl.BlockSpec((2, BR), lambda i: (0, i))
_XLSPEC = pl.BlockSpec((4, BR, 16), lambda i: (0, i, 0))


def _in_pre_body(x_ref, wt_ref, b_ref, gwt_ref, asrc_ref, adst_ref, hwt_ref,
                 hb_ref, xc_ref, xl_ref, s_ref, d_ref, hh_ref):
    xc = jnp.dot(x_ref[...], wt_ref[...],
                 preferred_element_type=F32) + b_ref[...]
    xc_ref[...] = xc
    _pre_part(xc, gwt_ref, asrc_ref, adst_ref, hwt_ref, hb_ref,
              xl_ref, s_ref, d_ref, hh_ref)


def _in_pre(x, wt, b, gwt, asrc, adst, hwt, hb):
    return pl.pallas_call(
        _in_pre_body,
        grid=(GRID,),
        in_specs=[
            pl.BlockSpec((BR, IN), lambda i: (i, 0)),
            pl.BlockSpec((IN, D), lambda i: (0, 0)),
            _VSPEC, _WSPEC, _VSPEC, _VSPEC, _WSPEC, _VSPEC,
        ],
        out_specs=[_MSPEC, _XLSPEC, _SSPEC, _SSPEC, _MSPEC],
        out_shape=[
            jax.ShapeDtypeStruct((NP, D), F32),
            jax.ShapeDtypeStruct((4, NP, 16), F32),
            jax.ShapeDtypeStruct((NP,), F32),
            jax.ShapeDtypeStruct((NP,), F32),
            jax.ShapeDtypeStruct((NP, D), F32),
        ],
    )(x, wt, b, gwt, asrc, adst, hwt, hb)


def _post_pre_body(conv_ref, dinv_ref, xc_ref, hh_ref, xloc_ref, lwt_ref,
                   lb_ref, gb_ref, lnw_ref, lnb_ref, beta_ref,
                   gwt_ref, asrc_ref, adst_ref, hwt_ref, hb_ref,
                   xcn_ref, xlocn_ref, xl_ref, s_ref, d_ref, hh2_ref):
    xcn, xlocn = _post_part(conv_ref, dinv_ref, xc_ref, hh_ref, xloc_ref,
                            lwt_ref, lb_ref, gb_ref, lnw_ref, lnb_ref,
                            beta_ref)
    xcn_ref[...] = xcn
    xlocn_ref[...] = xlocn
    _pre_part(xcn, gwt_ref, asrc_ref, adst_ref, hwt_ref, hb_ref,
              xl_ref, s_ref, d_ref, hh2_ref)


def _post_pre(conv, dinv, xc, hh, xloc, lwt, lb, gb, lnw, lnb, beta,
              gwt, asrc, adst, hwt, hb):
    return pl.pallas_call(
        _post_pre_body,
        grid=(GRID,),
        in_specs=[
            _XLSPEC, _DPSPEC, _MSPEC, _MSPEC, _MSPEC,
            _WSPEC, _VSPEC, _VSPEC, _VSPEC, _VSPEC, _VSPEC,
            _WSPEC, _VSPEC, _VSPEC, _WSPEC, _VSPEC,
        ],
        out_specs=[_MSPEC, _MSPEC, _XLSPEC, _SSPEC, _SSPEC, _MSPEC],
        out_shape=[
            jax.ShapeDtypeStruct((NP, D), F32),
            jax.ShapeDtypeStruct((NP, D), F32),
            jax.ShapeDtypeStruct((4, NP, 16), F32),
            jax.ShapeDtypeStruct((NP,), F32),
            jax.ShapeDtypeStruct((NP,), F32),
            jax.ShapeDtypeStruct((NP, D), F32),
        ],
    )(conv, dinv, xc, hh, xloc, lwt, lb, gb, lnw, lnb, beta,
      gwt, asrc, adst, hwt, hb)


def _post_pool_body(conv_ref, dinv_ref, xc_ref, hh_ref, xloc_ref, lwt_ref,
                    lb_ref, gb_ref, lnw_ref, lnb_ref, beta_ref,
                    batch_ref, pwt_ref, pb_ref, o_ref, acc_s, acc_c):
    pid = pl.program_id(0)

    @pl.when(pid == 0)
    def _():
        acc_s[...] = jnp.zeros((G, D), F32)
        acc_c[...] = jnp.zeros((G, D), F32)

    _, xlocn = _post_part(conv_ref, dinv_ref, xc_ref, hh_ref, xloc_ref,
                          lwt_ref, lb_ref, gb_ref, lnw_ref, lnb_ref,
                          beta_ref)
    b = batch_ref[...]
    gids = lax.broadcasted_iota(jnp.int32, (G, BR), 0)
    mask = (b[None, :] == gids).astype(F32)
    acc_s[...] += jnp.dot(mask, xlocn, preferred_element_type=F32)
    cnt = jnp.sum(mask, axis=1)
    acc_c[...] += jnp.broadcast_to(cnt[:, None], (G, D))

    @pl.when(pid == GRID - 1)
    def _():
        pooled = acc_s[...] / jnp.maximum(acc_c[...], 1.0)
        o_ref[...] = jnp.dot(pooled, pwt_ref[...],
                             preferred_element_type=F32) + pb_ref[...]


def _post_pool(conv, dinv, xc, hh, xloc, lwt, lb, gb, lnw, lnb, beta,
               batch_pad, pwt, pb):
    return pl.pallas_call(
        _post_pool_body,
        grid=(GRID,),
        in_specs=[
            _XLSPEC, _DPSPEC, _MSPEC, _MSPEC, _MSPEC,
            _WSPEC, _VSPEC, _VSPEC, _VSPEC, _VSPEC, _VSPEC,
            _SSPEC,
            pl.BlockSpec((D, G), lambda i: (0, 0)),
            pl.BlockSpec((G,), lambda i: (0,)),
        ],
        out_specs=pl.BlockSpec((G, G), lambda i: (0, 0)),
        out_shape=jax.ShapeDtypeStruct((G, G), F32),
        scratch_shapes=[pltpu.VMEM((G, D), F32), pltpu.VMEM((G, D), F32)],
    )(conv, dinv, xc, hh, xloc, lwt, lb, gb, lnw, lnb, beta,
      batch_pad, pwt, pb)


_NC = 2
_NS = 16
_CH = 128
_NPT = NP // _NS
_P1_PER = EP // (_NC * _NS)
_P1_CHUNKS = _P1_PER // _CH
_P2_PER = EP // _NS
_P2_CHUNKS = _P2_PER // _CH

_SCMESH = plsc.VectorSubcoreMesh(core_axis_name="c", subcore_axis_name="s",
                                 num_cores=_NC, num_subcores=_NS)
_K = 4
_K2 = 8


@functools.partial(
    pl.kernel,
    out_type=[jax.ShapeDtypeStruct((EP,), F32),
              jax.ShapeDtypeStruct((_NC, NP), F32)],
    mesh=_SCMESH,
    compiler_params=pltpu.CompilerParams(use_tc_tiling_on_sc=False),
    scratch_types=[
        pltpu.VMEM((2, _K * _CH), jnp.int32),
        pltpu.VMEM((2, _K * _CH), jnp.int32),
        pltpu.VMEM((2, _K * _CH), F32),
        pltpu.VMEM((2, _K * _CH), F32),
        pltpu.VMEM((2, _K * _CH), F32),
        pltpu.VMEM((_NPT,), F32),
        pltpu.VMEM_SHARED((NP,), F32),
        pltpu.SemaphoreType.DMA,
        pltpu.SemaphoreType.DMA,
        pltpu.SemaphoreType.DMA,
        pltpu.SemaphoreType.DMA,
        pltpu.SemaphoreType.DMA,
        pltpu.SemaphoreType.DMA,
        pltpu.SemaphoreType.DMA,
        pltpu.SemaphoreType.DMA,
        pltpu.SemaphoreType.DMA,
        pltpu.SemaphoreType.DMA,
        pltpu.SemaphoreType.DMA,
        pltpu.SemaphoreType.DMA,
    ])
def _edge_pass1(src_hbm, dst_hbm, s_hbm, d_hbm, w_hbm, dpart_hbm,
                srcb, dstb, sv, dv, wv, bounce, den_sh,
                semA0, semA1, sg00, sg01, sg02, sg03, sg10, sg11, sg12, sg13,
                semW0, semW1):
    cid = lax.axis_index("c")
    sid = lax.axis_index("s")
    wid = cid * _NS + sid
    nbase = sid * _NPT
    semA = [semA0, semA1]
    semG = [[sg00, sg01, sg02, sg03], [sg10, sg11, sg12, sg13]]
    semW = [semW0, semW1]

    zero = jnp.zeros((16,), F32)

    def zbody(i, _):
        bounce[pl.ds(i * 16, 16)] = zero
        return 0

    lax.fori_loop(0, _NPT // 16, zbody, 0)
    pltpu.sync_copy(bounce, den_sh.at[pl.ds(nbase, _NPT)])
    plsc.subcore_barrier()

    ebase0 = wid * _P1_PER
    n = _P1_CHUNKS // _K

    def issue_loads(k, s):
        gb = ebase0 + k * (_K * _CH)
        for j in range(_K):
            eb = gb + j * _CH
            pltpu.async_copy(src_hbm.at[pl.ds(eb, _CH)], srcb.at[s, pl.ds(j * _CH, _CH)],
                             semA[s])
            pltpu.async_copy(dst_hbm.at[pl.ds(eb, _CH)], dstb.at[s, pl.ds(j * _CH, _CH)],
                             semA[s])

    def drain_loads(k, s):
        gb = ebase0 + k * (_K * _CH)
        for j in range(_K):
            eb = gb + j * _CH
            pltpu.make_async_copy(src_hbm.at[pl.ds(eb, _CH)], srcb.at[s, pl.ds(j * _CH, _CH)],
                                  semA[s]).wait()
            pltpu.make_async_copy(dst_hbm.at[pl.ds(eb, _CH)], dstb.at[s, pl.ds(j * _CH, _CH)],
                                  semA[s]).wait()

    def issue_gathers(s):
        for j in range(_K):
            pltpu.async_copy(s_hbm.at[srcb.at[s, pl.ds(j * _CH, _CH)]], sv.at[s, pl.ds(j * _CH, _CH)], semG[s][j])
            pltpu.async_copy(d_hbm.at[dstb.at[s, pl.ds(j * _CH, _CH)]], dv.at[s, pl.ds(j * _CH, _CH)], semG[s][j])

    def drain_writes(k, s):
        gb = ebase0 + k * (_K * _CH)
        for j in range(_K):
            pltpu.make_async_copy(wv.at[s, pl.ds(j * _CH, _CH)],
                                  w_hbm.at[pl.ds(gb + j * _CH, _CH)],
                                  semW[s]).wait()

    def process(k, s):
        gb = ebase0 + k * (_K * _CH)

        @pl.when(k >= 2)
        def _():
            drain_writes(k - 2, s)

        for j in range(_K):
            pltpu.make_async_copy(s_hbm.at[srcb.at[s, pl.ds(j * _CH, _CH)]], sv.at[s, pl.ds(j * _CH, _CH)],
                                  semG[s][j]).wait()
            pltpu.make_async_copy(d_hbm.at[dstb.at[s, pl.ds(j * _CH, _CH)]], dv.at[s, pl.ds(j * _CH, _CH)],
                                  semG[s][j]).wait()
            for g in range(_CH // 16):
                a = sv[s, pl.ds(j * _CH + g * 16, 16)] + dv[s, pl.ds(j * _CH + g * 16, 16)]
                a = jnp.maximum(a, 0.0) + 0.2 * jnp.minimum(a, 0.0)
                wv[s, pl.ds(j * _CH + g * 16, 16)] = jnp.exp(a)
            pltpu.async_copy(wv.at[s, pl.ds(j * _CH, _CH)], w_hbm.at[pl.ds(gb + j * _CH, _CH)],
                             semW[s])
        pltpu.sync_copy(wv.at[s], den_sh.at[dstb.at[s]], add=True)

    issue_loads(0, 0)
    drain_loads(0, 0)
    issue_gathers(0)
    issue_loads(1, 1)

    def body(t, _):
        for h in range(2):
            k = 2 * t + h

            @pl.when(k + 1 < n)
            def _():
                drain_loads(k + 1, 1 - h)
                issue_gathers(1 - h)

            process(k, h)

            @pl.when(k + 2 < n)
            def _():
                issue_loads(k + 2, h)
        return 0

    lax.fori_loop(0, n // 2, body, 0)
    drain_writes(n - 2, 0)
    drain_writes(n - 1, 1)
    plsc.subcore_barrier()
    pltpu.sync_copy(den_sh.at[pl.ds(nbase, _NPT)], bounce)
    pltpu.sync_copy(bounce, dpart_hbm.at[cid].at[pl.ds(nbase, _NPT)])


@functools.partial(
    pl.kernel,
    out_type=jax.ShapeDtypeStruct((4 * NP, 16), F32),
    mesh=_SCMESH,
    compiler_params=pltpu.CompilerParams(use_tc_tiling_on_sc=False),
    scratch_types=[
        pltpu.VMEM((2, _K2 * _CH), jnp.int32),
        pltpu.VMEM((2, _K2 * _CH), jnp.int32),
        pltpu.VMEM((2, _K2 * _CH), jnp.int32),
        pltpu.VMEM((2, _K2 * _CH), F32),
        pltpu.VMEM((2, _K2 * _CH, 16), F32),
        pltpu.VMEM((2, _K2 * _CH, 16), F32),
        pltpu.VMEM_SHARED((NP, 16), F32),
        pltpu.SemaphoreType.DMA,
        pltpu.SemaphoreType.DMA,
        pltpu.SemaphoreType.DMA,
        pltpu.SemaphoreType.DMA,
    ])
def _edge_pass2(src_hbm, dst_hbm, w_hbm, zero_hbm, xl_hbm, out_hbm,
                srcb, dstb, srcb2, wv, rows, msg, acc_sh,
                semA0, semA1, sg0, sg1):
    cid = lax.axis_index("c")
    sid = lax.axis_index("s")
    nbase = sid * _NPT
    semA = [semA0, semA1]
    semG = [sg0, sg1]
    ebase0 = sid * _P2_PER
    n = _P2_CHUNKS // _K2

    def issue_loads(k, s):
        gb = ebase0 + k * (_K2 * _CH)
        for j in range(_K2):
            eb = gb + j * _CH
            pltpu.async_copy(src_hbm.at[pl.ds(eb, _CH)], srcb.at[s, pl.ds(j * _CH, _CH)],
                             semA[s])
            pltpu.async_copy(dst_hbm.at[pl.ds(eb, _CH)], dstb.at[s, pl.ds(j * _CH, _CH)],
                             semA[s])
            pltpu.async_copy(w_hbm.at[pl.ds(eb, _CH)], wv.at[s, pl.ds(j * _CH, _CH)], semA[s])

    def drain_loads(k, s):
        gb = ebase0 + k * (_K2 * _CH)
        for j in range(_K2):
            eb = gb + j * _CH
            pltpu.make_async_copy(src_hbm.at[pl.ds(eb, _CH)], srcb.at[s, pl.ds(j * _CH, _CH)],
                                  semA[s]).wait()
            pltpu.make_async_copy(dst_hbm.at[pl.ds(eb, _CH)], dstb.at[s, pl.ds(j * _CH, _CH)],
                                  semA[s]).wait()
            pltpu.make_async_copy(w_hbm.at[pl.ds(eb, _CH)], wv.at[s, pl.ds(j * _CH, _CH)],
                                  semA[s]).wait()

    def issue_gathers(s, rowoff):
        for j in range(_K2):
            for g in range(_CH // 16):
                srcb2[s, pl.ds(j * _CH + g * 16, 16)] = (
                    srcb[s, pl.ds(j * _CH + g * 16, 16)] + rowoff)
            pltpu.async_copy(xl_hbm.at[srcb2.at[s, pl.ds(j * _CH, _CH)]],
                             rows.at[s, pl.ds(j * _CH, _CH), :], semG[s])

    def process(k, s):
        for j in range(_K2):
            pltpu.make_async_copy(xl_hbm.at[srcb2.at[s, pl.ds(j * _CH, _CH)]],
                                  rows.at[s, pl.ds(j * _CH, _CH), :],
                                  semG[s]).wait()
        for j in range(_K2):
            for g in range(_CH // 16):
                attg = wv[s, pl.ds(j * _CH + g * 16, 16)]
                for t in range(16):
                    e = g * 16 + t
                    msg[s, j * _CH + e, pl.ds(0, 16)] = (
                        rows[s, j * _CH + e, pl.ds(0, 16)] * attg[t])
        pltpu.sync_copy(msg.at[s], acc_sh.at[dstb.at[s]], add=True)

    for sweep in range(2):
        rowoff = (2 * sweep + cid) * NP

        pltpu.sync_copy(zero_hbm, acc_sh.at[pl.ds(nbase, _NPT), :])
        plsc.subcore_barrier()

        issue_loads(0, 0)
        drain_loads(0, 0)
        issue_gathers(0, rowoff)
        issue_loads(1, 1)

        def body(t, _):
            for h in range(2):
                k = 2 * t + h

                @pl.when(k + 1 < n)
                def _():
                    drain_loads(k + 1, 1 - h)
                    issue_gathers(1 - h, rowoff)

                process(k, h)

                @pl.when(k + 2 < n)
                def _():
                    issue_loads(k + 2, h)
            return 0

        lax.fori_loop(0, n // 2, body, 0)
        plsc.subcore_barrier()
        pltpu.sync_copy(acc_sh.at[pl.ds(nbase, _NPT), :],
                        out_hbm.at[pl.ds(rowoff + nbase, _NPT), :])
        if sweep == 0:
            plsc.subcore_barrier()


_Z32 = None


def _gat_edges(src, dst, s, d, xl2):
    w, dpart = _edge_pass1(src, dst, s, d)
    zero16 = jnp.zeros((_NPT, 16), F32)
    out = _edge_pass2(src, dst, w, zero16, xl2.reshape(4 * NP, 16))
    return out.reshape(4, NP, 16), dpart


def kernel(x, edge_index, batch, params):
    src = jnp.concatenate([edge_index[0],
                           jnp.zeros((EP - E,), jnp.int32)])
    dst = jnp.concatenate([edge_index[1],
                           jnp.full((EP - E,), N, jnp.int32)])
    batch_pad = jnp.concatenate([batch, jnp.full((NP - N,), G, jnp.int32)])
    xp = jnp.pad(x, ((0, NP - N), (0, 0)))

    p0 = params['layers'][0]
    xc, xl2, s, d, hh = _in_pre(
        xp, params['lin_in_W'].T, params['lin_in_b'],
        p0['gat_W'].T, p0['att_src'][0, 0], p0['att_dst'][0, 0],
        p0['h_W'].T, p0['h_b'])
    xloc = jnp.zeros((NP, D), F32)
    for i in range(L):
        p = params['layers'][i]
        conv, dinv = _gat_edges(src, dst, s, d, xl2)
        if i < L - 1:
            pn = params['layers'][i + 1]
            xc, xloc, xl2, s, d, hh = _post_pre(
                conv, dinv, xc, hh, xloc, p['lin_W'].T, p['lin_b'],
                p['gat_b'], p['ln_w'], p['ln_b'], params['betas'][i],
                pn['gat_W'].T, pn['att_src'][0, 0], pn['att_dst'][0, 0],
                pn['h_W'].T, pn['h_b'])
        else:
            out = _post_pool(
                conv, dinv, xc, hh, xloc, p['lin_W'].T, p['lin_b'],
                p['gat_b'], p['ln_w'], p['ln_b'], params['betas'][i],
                batch_pad, params['pred_W'].T, params['pred_b'])
    return out
